# Initial kernel scaffold; baseline (speedup 1.0000x reference)
import jax
import jax.numpy as jnp
from jax.experimental import pallas as pl


def kernel(x, edge_index, e, Wn, bn, We, be, A, B, C, U, V, eb, nb, W1, b1, W2, b2):
    raise NotImplementedError("write your pallas kernel here")


# speedup vs baseline: 1.5686x; 1.5686x over previous
import functools

import jax
import jax.numpy as jnp
from jax import lax
from jax.experimental import pallas as pl
from jax.experimental.pallas import tpu as pltpu
from jax.experimental.pallas import tpu_sc as plsc

_N = 10000
_E = 320000
_D = 128
_DH = 64
_NS = 16
_CH = 40
_EPW = _E // _NS
_NCH = _EPW // _CH
_ZR = (_N // _NS) // 8 * 8
_ZREM = _N - _NS * _ZR

_BN = 2000
_BE = 2000


def _mm_kernel(x_ref, w_ref, b_ref, o_ref):
    o_ref[...] = (
        jnp.dot(x_ref[...], w_ref[...], preferred_element_type=jnp.float32)
        + b_ref[...]
    )


def _node_encode(x, Wn, bn):
    return pl.pallas_call(
        _mm_kernel,
        grid=(_N // _BN,),
        in_specs=[
            pl.BlockSpec((_BN, _D), lambda i: (i, 0)),
            pl.BlockSpec((_D, _D), lambda i: (0, 0)),
            pl.BlockSpec((1, _D), lambda i: (0, 0)),
        ],
        out_specs=pl.BlockSpec((_BN, _D), lambda i: (i, 0)),
        out_shape=jax.ShapeDtypeStruct((_N, _D), jnp.float32),
    )(x, Wn, bn.reshape(1, _D))


def _split_mm_kernel(x_ref, w_ref, b_ref, o_ref):
    y = (
        jnp.dot(x_ref[...], w_ref[...], preferred_element_type=jnp.float32)
        + b_ref[...]
    )
    o_ref[0] = y[:, :_DH]
    o_ref[1] = y[:, _DH:]


def _edge_encode(e, We, be):
    de = e.shape[1]
    return pl.pallas_call(
        _split_mm_kernel,
        grid=(_E // _BE,),
        in_specs=[
            pl.BlockSpec((_BE, de), lambda i: (i, 0)),
            pl.BlockSpec((de, _D), lambda i: (0, 0)),
            pl.BlockSpec((1, _D), lambda i: (0, 0)),
        ],
        out_specs=pl.BlockSpec((2, _BE, _DH), lambda i: (0, i, 0)),
        out_shape=jax.ShapeDtypeStruct((2, _E, _DH), jnp.float32),
    )(e, We, be.reshape(1, _D))


def _edge_mat_kernel(ef_ref, w_ref, b_ref, o_ref):
    xx = jnp.concatenate([ef_ref[0], ef_ref[1]], axis=1)
    y = jnp.dot(xx, w_ref[...], preferred_element_type=jnp.float32) + b_ref[...]
    o_ref[0] = y[:, :_DH]
    o_ref[1] = y[:, _DH:]


def _edge_mat(ef2, W, b):
    return pl.pallas_call(
        _edge_mat_kernel,
        grid=(_E // _BE,),
        in_specs=[
            pl.BlockSpec((2, _BE, _DH), lambda i: (0, i, 0)),
            pl.BlockSpec((_D, _D), lambda i: (0, 0)),
            pl.BlockSpec((1, _D), lambda i: (0, 0)),
        ],
        out_specs=pl.BlockSpec((2, _BE, _DH), lambda i: (0, i, 0)),
        out_shape=jax.ShapeDtypeStruct((2, _E, _DH), jnp.float32),
    )(ef2, W, b.reshape(1, _D))


def _node_mats_kernel(h_ref, a_ref, v_ref, bc_ref, bo_ref, ts_ref, td_ref):
    h = h_ref[...]
    ha = jnp.dot(h, a_ref[0], preferred_element_type=jnp.float32)
    hv = jnp.dot(h, v_ref[0], preferred_element_type=jnp.float32)
    ts_ref[...] = jnp.concatenate([ha, hv], axis=1)
    hbc = jnp.dot(h, bc_ref[0], preferred_element_type=jnp.float32)
    hbo = jnp.dot(h, bo_ref[0], preferred_element_type=jnp.float32)
    td_ref[...] = jnp.concatenate([hbc, hbo], axis=1)


def _halves(W):
    return W.reshape(_D, 2, _DH).transpose(1, 0, 2)


def _node_mats(h, Al, Vl, Bl):
    nb = _N // _BN
    return pl.pallas_call(
        _node_mats_kernel,
        grid=(2, nb),
        in_specs=[
            pl.BlockSpec((_BN, _D), lambda c, i: (i, 0)),
            pl.BlockSpec((1, _D, _DH), lambda c, i: (c, 0, 0)),
            pl.BlockSpec((1, _D, _DH), lambda c, i: (c, 0, 0)),
            pl.BlockSpec((1, _D, _DH), lambda c, i: (c, 0, 0)),
            pl.BlockSpec((1, _D, _DH), lambda c, i: (1 - c, 0, 0)),
        ],
        out_specs=[
            pl.BlockSpec((_BN, _D), lambda c, i: (c * nb + i, 0)),
            pl.BlockSpec((_BN, _D), lambda c, i: (c * nb + i, 0)),
        ],
        out_shape=[
            jax.ShapeDtypeStruct((2 * _N, _D), jnp.float32),
            jax.ShapeDtypeStruct((2 * _N, _D), jnp.float32),
        ],
    )(h, _halves(Al), _halves(Vl), _halves(Bl), _halves(Bl))


def _tables2_kernel(h_ref, wa_ref, wb_ref, o_ref):
    h = h_ref[...]
    hs = jnp.dot(h, wa_ref[0], preferred_element_type=jnp.float32)
    hd = jnp.dot(h, wb_ref[0], preferred_element_type=jnp.float32)
    o_ref[...] = jnp.concatenate([hs, hd], axis=1)


def _tables2(h, Wa, Wb):
    nb = _N // _BN
    return pl.pallas_call(
        _tables2_kernel,
        grid=(2, nb),
        in_specs=[
            pl.BlockSpec((_BN, _D), lambda c, i: (i, 0)),
            pl.BlockSpec((1, _D, _DH), lambda c, i: (c, 0, 0)),
            pl.BlockSpec((1, _D, _DH), lambda c, i: (c, 0, 0)),
        ],
        out_specs=pl.BlockSpec((_BN, _D), lambda c, i: (c * nb + i, 0)),
        out_shape=jax.ShapeDtypeStruct((2 * _N, _D), jnp.float32),
    )(h, _halves(Wa), _halves(Wb))


def _node_update_kernel(h_ref, u_ref, nb_ref, acc_ref, o_ref):
    h = h_ref[...]
    a0 = acc_ref[0]
    a1 = acc_ref[1]
    agg = jnp.concatenate([a0[:, :_DH], a1[:, :_DH]], axis=1)
    den = jnp.concatenate([a0[:, _DH:], a1[:, _DH:]], axis=1) + 1e-6
    hu = jnp.dot(h, u_ref[...], preferred_element_type=jnp.float32)
    o_ref[...] = h + jnp.maximum(hu + nb_ref[...] + agg / den, 0.0)


def _node_update(h, Ul, nbl, acc3):
    return pl.pallas_call(
        _node_update_kernel,
        grid=(_N // _BN,),
        in_specs=[
            pl.BlockSpec((_BN, _D), lambda i: (i, 0)),
            pl.BlockSpec((_D, _D), lambda i: (0, 0)),
            pl.BlockSpec((1, _D), lambda i: (0, 0)),
            pl.BlockSpec((2, _BN, _D), lambda i: (0, i, 0)),
        ],
        out_specs=pl.BlockSpec((_BN, _D), lambda i: (i, 0)),
        out_shape=jax.ShapeDtypeStruct((_N, _D), jnp.float32),
    )(h, Ul, nbl.reshape(1, _D), acc3)


def _combine_kernel(p_ref, b_ref, o_ref):
    s = p_ref[0] + p_ref[1]
    o_ref[...] = jnp.sum(s, axis=1, keepdims=True) + b_ref[...]


def _combine(part3, b2sc):
    return pl.pallas_call(
        _combine_kernel,
        grid=(_E // _BE,),
        in_specs=[
            pl.BlockSpec((2, _BE, 16), lambda i: (0, i, 0)),
            pl.BlockSpec((1, 1), lambda i: (0, 0)),
        ],
        out_specs=pl.BlockSpec((_BE, 1), lambda i: (i, 0)),
        out_shape=jax.ShapeDtypeStruct((_E, 1), jnp.float32),
    )(part3, b2sc)


def _sc_edge_body(dst_h, src2_h, dst2_h, zeros_h, tsrc_h, tdst_h, ef_h, efc_h,
                  efo_h, acc_h,
                  srcv, dstv, didx, gsrc, gdst, efv, efcv, scat, efov,
                  accsh, sem0, sem1, sem2, sem3):
    c = lax.axis_index("c")
    s = lax.axis_index("s")
    cN = c * _N
    cE = c * _E
    base = s * _EPW

    pltpu.sync_copy(zeros_h.at[pl.ds(s * _ZR, _ZR)],
                    accsh.at[pl.ds(s * _ZR, _ZR)])

    @pl.when(s == 0)
    def _zero_rem():
        pltpu.sync_copy(zeros_h.at[pl.ds(_NS * _ZR, _ZREM)],
                        accsh.at[pl.ds(_NS * _ZR, _ZREM)])

    plsc.subcore_barrier()

    def chunk(k, carry):
        off = pl.multiple_of(base + k * _CH, 8)
        pltpu.sync_copy(src2_h.at[pl.ds(cE + off, _CH)], srcv)
        pltpu.sync_copy(dst2_h.at[pl.ds(cE + off, _CH)], didx)
        pltpu.sync_copy(dst_h.at[pl.ds(off, _CH)], dstv)
        cp0 = pltpu.async_copy(tsrc_h.at[srcv], gsrc, sem0)
        cp1 = pltpu.async_copy(tdst_h.at[didx], gdst, sem1)
        cp2 = pltpu.async_copy(ef_h.at[pl.ds(cE + off, _CH)], efv, sem2)
        cp3 = pltpu.async_copy(efc_h.at[pl.ds(cE + off, _CH)], efcv, sem3)
        cp0.wait()
        cp1.wait()
        cp2.wait()
        cp3.wait()

        def row(r, carry2):
            for j in range(_DH // 16):
                sl = pl.ds(j * 16, 16)
                slv = pl.ds(_DH + j * 16, 16)
                a_ = gsrc[r, sl]
                v_ = gsrc[r, slv]
                b_ = gdst[r, sl]
                f_ = efv[r, sl]
                t_ = efcv[r, sl]
                fn = f_ + jnp.maximum(a_ + b_ + t_, 0.0)
                efov[r, sl] = fn
                eta = 1.0 / (1.0 + jnp.exp(-fn))
                scat[r, slv] = eta
                scat[r, sl] = eta * v_
            return carry2

        lax.fori_loop(0, _CH, row, 0)
        pltpu.sync_copy(efov, efo_h.at[pl.ds(cE + off, _CH)])
        pltpu.sync_copy(scat, accsh.at[dstv], add=True)
        return carry

    lax.fori_loop(0, _NCH, chunk, 0)
    plsc.subcore_barrier()
    pltpu.sync_copy(accsh.at[pl.ds(s * _ZR, _ZR)],
                    acc_h.at[pl.ds(cN + s * _ZR, _ZR)])

    @pl.when(s == 0)
    def _wb_rem():
        pltpu.sync_copy(accsh.at[pl.ds(_NS * _ZR, _ZREM)],
                        acc_h.at[pl.ds(cN + _NS * _ZR, _ZREM)])


@functools.cache
def _sc_edge():
    return pl.kernel(
    _sc_edge_body,
    out_type=[
        jax.ShapeDtypeStruct((2 * _E, _DH), jnp.float32),
        jax.ShapeDtypeStruct((2 * _N, _D), jnp.float32),
    ],
    mesh=plsc.VectorSubcoreMesh(core_axis_name="c", subcore_axis_name="s",
                                num_cores=2, num_subcores=_NS),
    compiler_params=pltpu.CompilerParams(needs_layout_passes=False),
    scratch_types=[
        pltpu.VMEM((_CH,), jnp.int32),
        pltpu.VMEM((_CH,), jnp.int32),
        pltpu.VMEM((_CH,), jnp.int32),
        pltpu.VMEM((_CH, _D), jnp.float32),
        pltpu.VMEM((_CH, _D), jnp.float32),
        pltpu.VMEM((_CH, _DH), jnp.float32),
        pltpu.VMEM((_CH, _DH), jnp.float32),
        pltpu.VMEM((_CH, _D), jnp.float32),
        pltpu.VMEM((_CH, _DH), jnp.float32),
        pltpu.VMEM_SHARED((_N, _D), jnp.float32),
        pltpu.SemaphoreType.DMA,
        pltpu.SemaphoreType.DMA,
        pltpu.SemaphoreType.DMA,
        pltpu.SemaphoreType.DMA,
    ],
    )


def _sc_score_body(src2_h, dst2_h, ts_h, efw_h, w2_h,
                   part_h,
                   srcv, dstv, g1, g2, efwv, w2v, outv, sem0, sem1, sem2):
    c = lax.axis_index("c")
    s = lax.axis_index("s")
    cE = c * _E
    base = s * _EPW
    pltpu.sync_copy(w2_h.at[c], w2v)

    def chunk(k, carry):
        off = pl.multiple_of(base + k * _CH, 8)
        pltpu.sync_copy(src2_h.at[pl.ds(cE + off, _CH)], srcv)
        pltpu.sync_copy(dst2_h.at[pl.ds(cE + off, _CH)], dstv)
        cp0 = pltpu.async_copy(ts_h.at[srcv], g1, sem0)
        cp1 = pltpu.async_copy(ts_h.at[dstv], g2, sem1)
        cp2 = pltpu.async_copy(efw_h.at[pl.ds(cE + off, _CH)], efwv, sem2)
        cp0.wait()
        cp1.wait()
        cp2.wait()

        def row(r, carry2):
            acc = jnp.zeros((16,), jnp.float32)
            for j in range(_DH // 16):
                sl = pl.ds(j * 16, 16)
                z = (g1[r, sl] + g2[r, pl.ds(_DH + j * 16, 16)]
                     + efwv[r, sl])
                acc = acc + jnp.maximum(z, 0.0) * w2v[0, sl]
            outv[r] = acc
            return carry2

        lax.fori_loop(0, _CH, row, 0)
        pltpu.sync_copy(outv, part_h.at[pl.ds(cE + off, _CH)])
        return carry

    lax.fori_loop(0, _NCH, chunk, 0)


@functools.cache
def _sc_score():
    return pl.kernel(
    _sc_score_body,
    out_type=jax.ShapeDtypeStruct((2 * _E, 16), jnp.float32),
    mesh=plsc.VectorSubcoreMesh(core_axis_name="c", subcore_axis_name="s",
                                num_cores=2, num_subcores=_NS),
    compiler_params=pltpu.CompilerParams(needs_layout_passes=False),
    scratch_types=[
        pltpu.VMEM((_CH,), jnp.int32),
        pltpu.VMEM((_CH,), jnp.int32),
        pltpu.VMEM((_CH, _D), jnp.float32),
        pltpu.VMEM((_CH, _D), jnp.float32),
        pltpu.VMEM((_CH, _DH), jnp.float32),
        pltpu.VMEM((1, _DH), jnp.float32),
        pltpu.VMEM((_CH, 16), jnp.float32),
        pltpu.SemaphoreType.DMA,
        pltpu.SemaphoreType.DMA,
        pltpu.SemaphoreType.DMA,
    ],
    )


def kernel(x, edge_index, e, Wn, bn, We, be, A, B, C, U, V, eb, nb,
           W1, b1, W2, b2):
    src = edge_index[0]
    dst = edge_index[1]
    src2 = jnp.concatenate([src, src + _N])
    dst2 = jnp.concatenate([dst, dst + _N])
    zeros = jnp.zeros((_N, _D), jnp.float32)
    h = _node_encode(x, Wn, bn)
    ef = _edge_encode(e, We, be)
    for l in range(A.shape[0]):
        tsrc, tdst = _node_mats(h, A[l], V[l], B[l])
        efc = _edge_mat(ef, C[l], eb[l])
        efo, acc = _sc_edge()(dst, src2, dst2, zeros, tsrc, tdst,
                              ef.reshape(2 * _E, _DH),
                              efc.reshape(2 * _E, _DH))
        ef = efo.reshape(2, _E, _DH)
        h = _node_update(h, U[l], nb[l], acc.reshape(2, _N, _D))
    ts = _tables2(h, W1[:_D], W1[_D:2 * _D])
    efw = _edge_mat(ef, W1[2 * _D:], b1)
    part = _sc_score()(src2, dst2, ts, efw.reshape(2 * _E, _DH),
                       W2[:, 0].reshape(2, 1, _DH))
    comb = _combine(part.reshape(2, _E, 16), b2.reshape(1, 1))
    return comb.reshape(_E)


# speedup vs baseline: 2.2598x; 1.4406x over previous
import functools

import jax
import jax.numpy as jnp
from jax import lax
from jax.experimental import pallas as pl
from jax.experimental.pallas import tpu as pltpu
from jax.experimental.pallas import tpu_sc as plsc

_N = 10000
_E = 320000
_D = 128
_DH = 64
_NS = 16
_CH = 40
_EPW = _E // _NS
_NCH = _EPW // _CH
_ZR = (_N // _NS) // 8 * 8
_ZREM = _N - _NS * _ZR

_BN = 2000
_BE = 2000


def _mm_kernel(x_ref, w_ref, b_ref, o_ref):
    o_ref[...] = (
        jnp.dot(x_ref[...], w_ref[...], preferred_element_type=jnp.float32)
        + b_ref[...]
    )


def _node_encode(x, Wn, bn):
    return pl.pallas_call(
        _mm_kernel,
        grid=(_N // _BN,),
        in_specs=[
            pl.BlockSpec((_BN, _D), lambda i: (i, 0)),
            pl.BlockSpec((_D, _D), lambda i: (0, 0)),
            pl.BlockSpec((1, _D), lambda i: (0, 0)),
        ],
        out_specs=pl.BlockSpec((_BN, _D), lambda i: (i, 0)),
        out_shape=jax.ShapeDtypeStruct((_N, _D), jnp.float32),
    )(x, Wn, bn.reshape(1, _D))


def _split_mm_kernel(x_ref, w_ref, b_ref, o_ref):
    y = (
        jnp.dot(x_ref[...], w_ref[...], preferred_element_type=jnp.float32)
        + b_ref[...]
    )
    o_ref[0] = y[:, :_DH]
    o_ref[1] = y[:, _DH:]


def _edge_encode(e, We, be):
    de = e.shape[1]
    return pl.pallas_call(
        _split_mm_kernel,
        grid=(_E // _BE,),
        in_specs=[
            pl.BlockSpec((_BE, de), lambda i: (i, 0)),
            pl.BlockSpec((de, _D), lambda i: (0, 0)),
            pl.BlockSpec((1, _D), lambda i: (0, 0)),
        ],
        out_specs=pl.BlockSpec((2, _BE, _DH), lambda i: (0, i, 0)),
        out_shape=jax.ShapeDtypeStruct((2, _E, _DH), jnp.float32),
    )(e, We, be.reshape(1, _D))


def _edge_mat_kernel(ef_ref, w_ref, b_ref, o_ref):
    xx = jnp.concatenate([ef_ref[0], ef_ref[1]], axis=1)
    y = jnp.dot(xx, w_ref[...], preferred_element_type=jnp.float32) + b_ref[...]
    o_ref[0] = y[:, :_DH]
    o_ref[1] = y[:, _DH:]


def _edge_mat(ef2, W, b):
    return pl.pallas_call(
        _edge_mat_kernel,
        grid=(_E // _BE,),
        in_specs=[
            pl.BlockSpec((2, _BE, _DH), lambda i: (0, i, 0)),
            pl.BlockSpec((_D, _D), lambda i: (0, 0)),
            pl.BlockSpec((1, _D), lambda i: (0, 0)),
        ],
        out_specs=pl.BlockSpec((2, _BE, _DH), lambda i: (0, i, 0)),
        out_shape=jax.ShapeDtypeStruct((2, _E, _DH), jnp.float32),
    )(ef2, W, b.reshape(1, _D))


def _node_mats_kernel(h_ref, a_ref, v_ref, bc_ref, bo_ref, ts_ref, td_ref):
    h = h_ref[...]
    ha = jnp.dot(h, a_ref[0], preferred_element_type=jnp.float32)
    hv = jnp.dot(h, v_ref[0], preferred_element_type=jnp.float32)
    ts_ref[...] = jnp.concatenate([ha, hv], axis=1)
    hbc = jnp.dot(h, bc_ref[0], preferred_element_type=jnp.float32)
    hbo = jnp.dot(h, bo_ref[0], preferred_element_type=jnp.float32)
    td_ref[...] = jnp.concatenate([hbc, hbo], axis=1)


def _halves(W):
    return W.reshape(_D, 2, _DH).transpose(1, 0, 2)


def _node_mats(h, Al, Vl, Bl):
    nb = _N // _BN
    return pl.pallas_call(
        _node_mats_kernel,
        grid=(2, nb),
        in_specs=[
            pl.BlockSpec((_BN, _D), lambda c, i: (i, 0)),
            pl.BlockSpec((1, _D, _DH), lambda c, i: (c, 0, 0)),
            pl.BlockSpec((1, _D, _DH), lambda c, i: (c, 0, 0)),
            pl.BlockSpec((1, _D, _DH), lambda c, i: (c, 0, 0)),
            pl.BlockSpec((1, _D, _DH), lambda c, i: (1 - c, 0, 0)),
        ],
        out_specs=[
            pl.BlockSpec((_BN, _D), lambda c, i: (c * nb + i, 0)),
            pl.BlockSpec((_BN, _D), lambda c, i: (c * nb + i, 0)),
        ],
        out_shape=[
            jax.ShapeDtypeStruct((2 * _N, _D), jnp.float32),
            jax.ShapeDtypeStruct((2 * _N, _D), jnp.float32),
        ],
    )(h, _halves(Al), _halves(Vl), _halves(Bl), _halves(Bl))


def _tables2_kernel(h_ref, wa_ref, wb_ref, o_ref):
    h = h_ref[...]
    hs = jnp.dot(h, wa_ref[0], preferred_element_type=jnp.float32)
    hd = jnp.dot(h, wb_ref[0], preferred_element_type=jnp.float32)
    o_ref[...] = jnp.concatenate([hs, hd], axis=1)


def _tables2(h, Wa, Wb):
    nb = _N // _BN
    return pl.pallas_call(
        _tables2_kernel,
        grid=(2, nb),
        in_specs=[
            pl.BlockSpec((_BN, _D), lambda c, i: (i, 0)),
            pl.BlockSpec((1, _D, _DH), lambda c, i: (c, 0, 0)),
            pl.BlockSpec((1, _D, _DH), lambda c, i: (c, 0, 0)),
        ],
        out_specs=pl.BlockSpec((_BN, _D), lambda c, i: (c * nb + i, 0)),
        out_shape=jax.ShapeDtypeStruct((2 * _N, _D), jnp.float32),
    )(h, _halves(Wa), _halves(Wb))


def _node_update_kernel(h_ref, u_ref, nb_ref, acc_ref, o_ref):
    h = h_ref[...]
    a0 = acc_ref[0]
    a1 = acc_ref[1]
    agg = jnp.concatenate([a0[:, :_DH], a1[:, :_DH]], axis=1)
    den = jnp.concatenate([a0[:, _DH:], a1[:, _DH:]], axis=1) + 1e-6
    hu = jnp.dot(h, u_ref[...], preferred_element_type=jnp.float32)
    o_ref[...] = h + jnp.maximum(hu + nb_ref[...] + agg / den, 0.0)


def _node_update(h, Ul, nbl, acc3):
    return pl.pallas_call(
        _node_update_kernel,
        grid=(_N // _BN,),
        in_specs=[
            pl.BlockSpec((_BN, _D), lambda i: (i, 0)),
            pl.BlockSpec((_D, _D), lambda i: (0, 0)),
            pl.BlockSpec((1, _D), lambda i: (0, 0)),
            pl.BlockSpec((2, _BN, _D), lambda i: (0, i, 0)),
        ],
        out_specs=pl.BlockSpec((_BN, _D), lambda i: (i, 0)),
        out_shape=jax.ShapeDtypeStruct((_N, _D), jnp.float32),
    )(h, Ul, nbl.reshape(1, _D), acc3)


def _combine_kernel(p_ref, b_ref, o_ref):
    s = p_ref[0] + p_ref[1]
    o_ref[...] = jnp.sum(s, axis=1, keepdims=True) + b_ref[...]


def _combine(part3, b2sc):
    return pl.pallas_call(
        _combine_kernel,
        grid=(_E // _BE,),
        in_specs=[
            pl.BlockSpec((2, _BE, 16), lambda i: (0, i, 0)),
            pl.BlockSpec((1, 1), lambda i: (0, 0)),
        ],
        out_specs=pl.BlockSpec((_BE, 1), lambda i: (i, 0)),
        out_shape=jax.ShapeDtypeStruct((_E, 1), jnp.float32),
    )(part3, b2sc)


def _sc_edge_body(dst_h, src2_h, dst2_h, zeros_h, tsrc_h, tdst_h, ef_h, efc_h,
                  efo_h, acc_h,
                  srcv, dstv, didx, gsrc, gdst, efv, efcv, scat,
                  accsh, sem0, sem1):
    c = lax.axis_index("c")
    s = lax.axis_index("s")
    cN = c * _N
    cE = c * _E
    base = s * _EPW
    sems = (sem0, sem1)

    pltpu.sync_copy(zeros_h.at[pl.ds(s * _ZR, _ZR)],
                    accsh.at[pl.ds(s * _ZR, _ZR)])

    @pl.when(s == 0)
    def _zero_rem():
        pltpu.sync_copy(zeros_h.at[pl.ds(_NS * _ZR, _ZREM)],
                        accsh.at[pl.ds(_NS * _ZR, _ZREM)])

    plsc.subcore_barrier()

    def _issue(b, k):
        off = pl.multiple_of(base + k * _CH, 8)
        pltpu.sync_copy(src2_h.at[pl.ds(cE + off, _CH)], srcv.at[b])
        pltpu.sync_copy(dst2_h.at[pl.ds(cE + off, _CH)], didx.at[b])
        pltpu.sync_copy(dst_h.at[pl.ds(off, _CH)], dstv.at[b])
        pltpu.async_copy(tsrc_h.at[srcv.at[b]], gsrc.at[b], sems[b])
        pltpu.async_copy(tdst_h.at[didx.at[b]], gdst.at[b], sems[b])
        pltpu.async_copy(ef_h.at[pl.ds(cE + off, _CH)], efv.at[b], sems[b])
        pltpu.async_copy(efc_h.at[pl.ds(cE + off, _CH)], efcv.at[b], sems[b])

    def _wait(b, k):
        off = pl.multiple_of(base + k * _CH, 8)
        pltpu.make_async_copy(tsrc_h.at[srcv.at[b]], gsrc.at[b], sems[b]).wait()
        pltpu.make_async_copy(tdst_h.at[didx.at[b]], gdst.at[b], sems[b]).wait()
        pltpu.make_async_copy(ef_h.at[pl.ds(cE + off, _CH)], efv.at[b],
                              sems[b]).wait()
        pltpu.make_async_copy(efc_h.at[pl.ds(cE + off, _CH)], efcv.at[b],
                              sems[b]).wait()

    _issue(0, 0)

    def outer(k2, carry):
        for b in range(2):
            k = k2 * 2 + b

            @pl.when(k + 1 < _NCH)
            def _prefetch():
                _issue(1 - b, k + 1)

            _wait(b, k)

            def row(r, carry2):
                for j in range(_DH // 16):
                    sl = pl.ds(j * 16, 16)
                    slv = pl.ds(_DH + j * 16, 16)
                    a_ = gsrc[b, r, sl]
                    v_ = gsrc[b, r, slv]
                    b_ = gdst[b, r, sl]
                    f_ = efv[b, r, sl]
                    t_ = efcv[b, r, sl]
                    fn = f_ + jnp.maximum(a_ + b_ + t_, 0.0)
                    efv[b, r, sl] = fn
                    eta = 1.0 / (1.0 + jnp.exp(-fn))
                    scat[r, slv] = eta
                    scat[r, sl] = eta * v_
                return carry2

            lax.fori_loop(0, _CH, row, 0)
            off = pl.multiple_of(base + k * _CH, 8)
            pltpu.sync_copy(efv.at[b], efo_h.at[pl.ds(cE + off, _CH)])
            pltpu.sync_copy(scat, accsh.at[dstv.at[b]], add=True)
        return carry

    lax.fori_loop(0, _NCH // 2, outer, 0)
    plsc.subcore_barrier()
    pltpu.sync_copy(accsh.at[pl.ds(s * _ZR, _ZR)],
                    acc_h.at[pl.ds(cN + s * _ZR, _ZR)])

    @pl.when(s == 0)
    def _wb_rem():
        pltpu.sync_copy(accsh.at[pl.ds(_NS * _ZR, _ZREM)],
                        acc_h.at[pl.ds(cN + _NS * _ZR, _ZREM)])


@functools.cache
def _sc_edge():
    return pl.kernel(
    _sc_edge_body,
    out_type=[
        jax.ShapeDtypeStruct((2 * _E, _DH), jnp.float32),
        jax.ShapeDtypeStruct((2 * _N, _D), jnp.float32),
    ],
    mesh=plsc.VectorSubcoreMesh(core_axis_name="c", subcore_axis_name="s",
                                num_cores=2, num_subcores=_NS),
    compiler_params=pltpu.CompilerParams(needs_layout_passes=False),
    scratch_types=[
        pltpu.VMEM((2, _CH), jnp.int32),
        pltpu.VMEM((2, _CH), jnp.int32),
        pltpu.VMEM((2, _CH), jnp.int32),
        pltpu.VMEM((2, _CH, _D), jnp.float32),
        pltpu.VMEM((2, _CH, _D), jnp.float32),
        pltpu.VMEM((2, _CH, _DH), jnp.float32),
        pltpu.VMEM((2, _CH, _DH), jnp.float32),
        pltpu.VMEM((_CH, _D), jnp.float32),
        pltpu.VMEM_SHARED((_N, _D), jnp.float32),
        pltpu.SemaphoreType.DMA,
        pltpu.SemaphoreType.DMA,
    ],
    )


def _sc_score_body(src2_h, dst2_h, ts_h, efw_h, w2_h,
                   part_h,
                   srcv, dstv, g1, g2, efwv, w2v, outv, sem0, sem1):
    c = lax.axis_index("c")
    s = lax.axis_index("s")
    cE = c * _E
    base = s * _EPW
    sems = (sem0, sem1)
    pltpu.sync_copy(w2_h.at[c], w2v)

    def _issue(b, k):
        off = pl.multiple_of(base + k * _CH, 8)
        pltpu.sync_copy(src2_h.at[pl.ds(cE + off, _CH)], srcv.at[b])
        pltpu.sync_copy(dst2_h.at[pl.ds(cE + off, _CH)], dstv.at[b])
        pltpu.async_copy(ts_h.at[srcv.at[b]], g1.at[b], sems[b])
        pltpu.async_copy(ts_h.at[dstv.at[b]], g2.at[b], sems[b])
        pltpu.async_copy(efw_h.at[pl.ds(cE + off, _CH)], efwv.at[b], sems[b])

    def _wait(b, k):
        off = pl.multiple_of(base + k * _CH, 8)
        pltpu.make_async_copy(ts_h.at[srcv.at[b]], g1.at[b], sems[b]).wait()
        pltpu.make_async_copy(ts_h.at[dstv.at[b]], g2.at[b], sems[b]).wait()
        pltpu.make_async_copy(efw_h.at[pl.ds(cE + off, _CH)], efwv.at[b],
                              sems[b]).wait()

    _issue(0, 0)

    def outer(k2, carry):
        for b in range(2):
            k = k2 * 2 + b

            @pl.when(k + 1 < _NCH)
            def _prefetch():
                _issue(1 - b, k + 1)

            _wait(b, k)

            def row(r, carry2):
                acc = jnp.zeros((16,), jnp.float32)
                for j in range(_DH // 16):
                    sl = pl.ds(j * 16, 16)
                    z = (g1[b, r, sl] + g2[b, r, pl.ds(_DH + j * 16, 16)]
                         + efwv[b, r, sl])
                    acc = acc + jnp.maximum(z, 0.0) * w2v[0, sl]
                outv[r] = acc
                return carry2

            lax.fori_loop(0, _CH, row, 0)
            off = pl.multiple_of(base + k * _CH, 8)
            pltpu.sync_copy(outv, part_h.at[pl.ds(cE + off, _CH)])
        return carry

    lax.fori_loop(0, _NCH // 2, outer, 0)


@functools.cache
def _sc_score():
    return pl.kernel(
    _sc_score_body,
    out_type=jax.ShapeDtypeStruct((2 * _E, 16), jnp.float32),
    mesh=plsc.VectorSubcoreMesh(core_axis_name="c", subcore_axis_name="s",
                                num_cores=2, num_subcores=_NS),
    compiler_params=pltpu.CompilerParams(needs_layout_passes=False),
    scratch_types=[
        pltpu.VMEM((2, _CH), jnp.int32),
        pltpu.VMEM((2, _CH), jnp.int32),
        pltpu.VMEM((2, _CH, _D), jnp.float32),
        pltpu.VMEM((2, _CH, _D), jnp.float32),
        pltpu.VMEM((2, _CH, _DH), jnp.float32),
        pltpu.VMEM((1, _DH), jnp.float32),
        pltpu.VMEM((_CH, 16), jnp.float32),
        pltpu.SemaphoreType.DMA,
        pltpu.SemaphoreType.DMA,
    ],
    )


def kernel(x, edge_index, e, Wn, bn, We, be, A, B, C, U, V, eb, nb,
           W1, b1, W2, b2):
    src = edge_index[0]
    dst = edge_index[1]
    src2 = jnp.concatenate([src, src + _N])
    dst2 = jnp.concatenate([dst, dst + _N])
    zeros = jnp.zeros((_N, _D), jnp.float32)
    h = _node_encode(x, Wn, bn)
    ef = _edge_encode(e, We, be)
    for l in range(A.shape[0]):
        tsrc, tdst = _node_mats(h, A[l], V[l], B[l])
        efc = _edge_mat(ef, C[l], eb[l])
        efo, acc = _sc_edge()(dst, src2, dst2, zeros, tsrc, tdst,
                              ef.reshape(2 * _E, _DH),
                              efc.reshape(2 * _E, _DH))
        ef = efo.reshape(2, _E, _DH)
        h = _node_update(h, U[l], nb[l], acc.reshape(2, _N, _D))
    ts = _tables2(h, W1[:_D], W1[_D:2 * _D])
    efw = _edge_mat(ef, W1[2 * _D:], b1)
    part = _sc_score()(src2, dst2, ts, efw.reshape(2 * _E, _DH),
                       W2[:, 0].reshape(2, 1, _DH))
    comb = _combine(part.reshape(2, _E, 16), b2.reshape(1, 1))
    return comb.reshape(_E)


# speedup vs baseline: 2.3916x; 1.0583x over previous
import functools

import jax
import jax.numpy as jnp
from jax import lax
from jax.experimental import pallas as pl
from jax.experimental.pallas import tpu as pltpu
from jax.experimental.pallas import tpu_sc as plsc

_N = 10000
_E = 320000
_D = 128
_DH = 64
_NS = 16
_CH = 40
_EPW = _E // _NS
_NCH = _EPW // _CH
_ZR = (_N // _NS) // 8 * 8
_ZREM = _N - _NS * _ZR
_SUP = 5
_NSUP = _NCH // _SUP

_BN = 2000
_BE = 2000


def _mm_kernel(x_ref, w_ref, b_ref, o_ref):
    o_ref[...] = (
        jnp.dot(x_ref[...], w_ref[...], preferred_element_type=jnp.float32)
        + b_ref[...]
    )


def _node_encode(x, Wn, bn):
    return pl.pallas_call(
        _mm_kernel,
        grid=(_N // _BN,),
        in_specs=[
            pl.BlockSpec((_BN, _D), lambda i: (i, 0)),
            pl.BlockSpec((_D, _D), lambda i: (0, 0)),
            pl.BlockSpec((1, _D), lambda i: (0, 0)),
        ],
        out_specs=pl.BlockSpec((_BN, _D), lambda i: (i, 0)),
        out_shape=jax.ShapeDtypeStruct((_N, _D), jnp.float32),
    )(x, Wn, bn.reshape(1, _D))


def _split_mm_kernel(x_ref, w_ref, b_ref, o_ref):
    y = (
        jnp.dot(x_ref[...], w_ref[...], preferred_element_type=jnp.float32)
        + b_ref[...]
    )
    o_ref[0] = y[:, :_DH]
    o_ref[1] = y[:, _DH:]


def _edge_encode(e, We, be):
    de = e.shape[1]
    return pl.pallas_call(
        _split_mm_kernel,
        grid=(_E // _BE,),
        in_specs=[
            pl.BlockSpec((_BE, de), lambda i: (i, 0)),
            pl.BlockSpec((de, _D), lambda i: (0, 0)),
            pl.BlockSpec((1, _D), lambda i: (0, 0)),
        ],
        out_specs=pl.BlockSpec((2, _BE, _DH), lambda i: (0, i, 0)),
        out_shape=jax.ShapeDtypeStruct((2, _E, _DH), jnp.float32),
    )(e, We, be.reshape(1, _D))


def _edge_mat_kernel(ef_ref, w_ref, b_ref, o_ref):
    xx = jnp.concatenate([ef_ref[0], ef_ref[1]], axis=1)
    y = jnp.dot(xx, w_ref[...], preferred_element_type=jnp.float32) + b_ref[...]
    o_ref[0] = y[:, :_DH]
    o_ref[1] = y[:, _DH:]


def _edge_mat(ef2, W, b):
    return pl.pallas_call(
        _edge_mat_kernel,
        grid=(_E // _BE,),
        in_specs=[
            pl.BlockSpec((2, _BE, _DH), lambda i: (0, i, 0)),
            pl.BlockSpec((_D, _D), lambda i: (0, 0)),
            pl.BlockSpec((1, _D), lambda i: (0, 0)),
        ],
        out_specs=pl.BlockSpec((2, _BE, _DH), lambda i: (0, i, 0)),
        out_shape=jax.ShapeDtypeStruct((2, _E, _DH), jnp.float32),
    )(ef2, W, b.reshape(1, _D))


def _edge_mat_packed_kernel(ef_ref, w_ref, b_ref, o_ref):
    xx = jnp.concatenate([ef_ref[0], ef_ref[1]], axis=1)
    y = jnp.dot(xx, w_ref[...], preferred_element_type=jnp.float32) + b_ref[...]
    o_ref[0] = jnp.concatenate([ef_ref[0], y[:, :_DH]], axis=1)
    o_ref[1] = jnp.concatenate([ef_ref[1], y[:, _DH:]], axis=1)


def _edge_mat_packed(ef2, W, b):
    return pl.pallas_call(
        _edge_mat_packed_kernel,
        grid=(_E // _BE,),
        in_specs=[
            pl.BlockSpec((2, _BE, _DH), lambda i: (0, i, 0)),
            pl.BlockSpec((_D, _D), lambda i: (0, 0)),
            pl.BlockSpec((1, _D), lambda i: (0, 0)),
        ],
        out_specs=pl.BlockSpec((2, _BE, _D), lambda i: (0, i, 0)),
        out_shape=jax.ShapeDtypeStruct((2, _E, _D), jnp.float32),
    )(ef2, W, b.reshape(1, _D))


def _node_mats_kernel(h_ref, a_ref, v_ref, bc_ref, bo_ref, ts_ref, td_ref):
    h = h_ref[...]
    ha = jnp.dot(h, a_ref[0], preferred_element_type=jnp.float32)
    hv = jnp.dot(h, v_ref[0], preferred_element_type=jnp.float32)
    ts_ref[...] = jnp.concatenate([ha, hv], axis=1)
    hbc = jnp.dot(h, bc_ref[0], preferred_element_type=jnp.float32)
    hbo = jnp.dot(h, bo_ref[0], preferred_element_type=jnp.float32)
    td_ref[...] = jnp.concatenate([hbc, hbo], axis=1)


def _halves(W):
    return W.reshape(_D, 2, _DH).transpose(1, 0, 2)


def _node_mats(h, Al, Vl, Bl):
    nb = _N // _BN
    return pl.pallas_call(
        _node_mats_kernel,
        grid=(2, nb),
        in_specs=[
            pl.BlockSpec((_BN, _D), lambda c, i: (i, 0)),
            pl.BlockSpec((1, _D, _DH), lambda c, i: (c, 0, 0)),
            pl.BlockSpec((1, _D, _DH), lambda c, i: (c, 0, 0)),
            pl.BlockSpec((1, _D, _DH), lambda c, i: (c, 0, 0)),
            pl.BlockSpec((1, _D, _DH), lambda c, i: (1 - c, 0, 0)),
        ],
        out_specs=[
            pl.BlockSpec((_BN, _D), lambda c, i: (c * nb + i, 0)),
            pl.BlockSpec((_BN, _D), lambda c, i: (c * nb + i, 0)),
        ],
        out_shape=[
            jax.ShapeDtypeStruct((2 * _N, _D), jnp.float32),
            jax.ShapeDtypeStruct((2 * _N, _D), jnp.float32),
        ],
    )(h, _halves(Al), _halves(Vl), _halves(Bl), _halves(Bl))


def _tables2_kernel(h_ref, wa_ref, wb_ref, o_ref):
    h = h_ref[...]
    hs = jnp.dot(h, wa_ref[0], preferred_element_type=jnp.float32)
    hd = jnp.dot(h, wb_ref[0], preferred_element_type=jnp.float32)
    o_ref[...] = jnp.concatenate([hs, hd], axis=1)


def _tables2(h, Wa, Wb):
    nb = _N // _BN
    return pl.pallas_call(
        _tables2_kernel,
        grid=(2, nb),
        in_specs=[
            pl.BlockSpec((_BN, _D), lambda c, i: (i, 0)),
            pl.BlockSpec((1, _D, _DH), lambda c, i: (c, 0, 0)),
            pl.BlockSpec((1, _D, _DH), lambda c, i: (c, 0, 0)),
        ],
        out_specs=pl.BlockSpec((_BN, _D), lambda c, i: (c * nb + i, 0)),
        out_shape=jax.ShapeDtypeStruct((2 * _N, _D), jnp.float32),
    )(h, _halves(Wa), _halves(Wb))


def _node_update_kernel(h_ref, u_ref, nb_ref, acc_ref, o_ref):
    h = h_ref[...]
    a0 = acc_ref[0]
    a1 = acc_ref[1]
    agg = jnp.concatenate([a0[:, :_DH], a1[:, :_DH]], axis=1)
    den = jnp.concatenate([a0[:, _DH:], a1[:, _DH:]], axis=1) + 1e-6
    hu = jnp.dot(h, u_ref[...], preferred_element_type=jnp.float32)
    o_ref[...] = h + jnp.maximum(hu + nb_ref[...] + agg / den, 0.0)


def _node_update(h, Ul, nbl, acc3):
    return pl.pallas_call(
        _node_update_kernel,
        grid=(_N // _BN,),
        in_specs=[
            pl.BlockSpec((_BN, _D), lambda i: (i, 0)),
            pl.BlockSpec((_D, _D), lambda i: (0, 0)),
            pl.BlockSpec((1, _D), lambda i: (0, 0)),
            pl.BlockSpec((2, _BN, _D), lambda i: (0, i, 0)),
        ],
        out_specs=pl.BlockSpec((_BN, _D), lambda i: (i, 0)),
        out_shape=jax.ShapeDtypeStruct((_N, _D), jnp.float32),
    )(h, Ul, nbl.reshape(1, _D), acc3)


def _combine_kernel(p_ref, b_ref, o_ref):
    s = p_ref[0] + p_ref[1]
    o_ref[...] = jnp.sum(s, axis=1, keepdims=True) + b_ref[...]


def _combine(part3, b2sc):
    return pl.pallas_call(
        _combine_kernel,
        grid=(_E // _BE,),
        in_specs=[
            pl.BlockSpec((2, _BE, 16), lambda i: (0, i, 0)),
            pl.BlockSpec((1, 1), lambda i: (0, 0)),
        ],
        out_specs=pl.BlockSpec((_BE, 1), lambda i: (i, 0)),
        out_shape=jax.ShapeDtypeStruct((_E, 1), jnp.float32),
    )(part3, b2sc)


def _sc_edge_body(dst_h, src2_h, dst2_h, zeros_h, tsrc_h, tdst_h, efx_h,
                  efo_h, acc_h,
                  ibs, ibd, ibr, gsrc, gdst, efx, scat, efov,
                  accsh, sem0, sem1, sem_eo, sem_sc):
    c = lax.axis_index("c")
    s = lax.axis_index("s")
    cN = c * _N
    cE = c * _E
    base = s * _EPW
    sups = s * _NSUP
    sems = (sem0, sem1)

    pltpu.sync_copy(zeros_h.at[pl.ds(s * _ZR, _ZR)],
                    accsh.at[pl.ds(s * _ZR, _ZR)])

    @pl.when(s == 0)
    def _zero_rem():
        pltpu.sync_copy(zeros_h.at[pl.ds(_NS * _ZR, _ZREM)],
                        accsh.at[pl.ds(_NS * _ZR, _ZREM)])

    plsc.subcore_barrier()

    def _issue(b, k):
        off = pl.multiple_of(base + k * _CH, 8)
        pltpu.sync_copy(src2_h.at[pl.ds(cE + off, _CH)], ibs.at[b])
        pltpu.sync_copy(dst2_h.at[pl.ds(cE + off, _CH)], ibd.at[b])
        pltpu.sync_copy(dst_h.at[pl.ds(off, _CH)], ibr.at[b])
        pltpu.async_copy(tsrc_h.at[ibs.at[b]], gsrc.at[b], sems[b])
        pltpu.async_copy(tdst_h.at[ibd.at[b]], gdst.at[b], sems[b])
        pltpu.async_copy(efx_h.at[pl.ds(cE + off, _CH)], efx.at[b], sems[b])

    def _wait(b, k):
        off = pl.multiple_of(base + k * _CH, 8)
        pltpu.make_async_copy(tsrc_h.at[ibs.at[b]], gsrc.at[b],
                              sems[b]).wait()
        pltpu.make_async_copy(tdst_h.at[ibd.at[b]], gdst.at[b],
                              sems[b]).wait()
        pltpu.make_async_copy(efx_h.at[pl.ds(cE + off, _CH)], efx.at[b],
                              sems[b]).wait()

    def _wait_out(kp):
        offp = pl.multiple_of(base + kp * _CH, 8)
        pltpu.make_async_copy(efov, efo_h.at[pl.ds(cE + offp, _CH)],
                              sem_eo).wait()

    _issue(0, 0)

    def outer(k2, carry):
        for b in range(2):
            k = k2 * 2 + b

            @pl.when(k + 1 < _NCH)
            def _prefetch():
                _issue(1 - b, k + 1)

            _wait(b, k)

            @pl.when(k > 0)
            def _dr():
                _wait_out(k - 1)

            def row(r, carry2):
                for jj in range(_DH // 16):
                    sl = pl.ds(jj * 16, 16)
                    slv = pl.ds(_DH + jj * 16, 16)
                    a_ = gsrc[b, r, sl]
                    v_ = gsrc[b, r, slv]
                    b_ = gdst[b, r, sl]
                    f_ = efx[b, r, sl]
                    t_ = efx[b, r, slv]
                    fn = f_ + jnp.maximum(a_ + b_ + t_, 0.0)
                    efov[r, sl] = fn
                    eta = 1.0 / (1.0 + jnp.exp(-fn))
                    scat[r, slv] = eta
                    scat[r, sl] = eta * v_
                return carry2

            lax.fori_loop(0, _CH, row, 0)
            off = pl.multiple_of(base + k * _CH, 8)
            pltpu.async_copy(efov, efo_h.at[pl.ds(cE + off, _CH)], sem_eo)
            pltpu.sync_copy(scat, accsh.at[ibr.at[b]], add=True)
        return carry

    lax.fori_loop(0, _NCH // 2, outer, 0)
    _wait_out(_NCH - 1)
    plsc.subcore_barrier()
    pltpu.sync_copy(accsh.at[pl.ds(s * _ZR, _ZR)],
                    acc_h.at[pl.ds(cN + s * _ZR, _ZR)])

    @pl.when(s == 0)
    def _wb_rem():
        pltpu.sync_copy(accsh.at[pl.ds(_NS * _ZR, _ZREM)],
                        acc_h.at[pl.ds(cN + _NS * _ZR, _ZREM)])


@functools.cache
def _sc_edge():
    return pl.kernel(
    _sc_edge_body,
    out_type=[
        jax.ShapeDtypeStruct((2 * _E, _DH), jnp.float32),
        jax.ShapeDtypeStruct((2 * _N, _D), jnp.float32),
    ],
    mesh=plsc.VectorSubcoreMesh(core_axis_name="c", subcore_axis_name="s",
                                num_cores=2, num_subcores=_NS),
    compiler_params=pltpu.CompilerParams(needs_layout_passes=False),
    scratch_types=[
        pltpu.VMEM((2, _CH), jnp.int32),
        pltpu.VMEM((2, _CH), jnp.int32),
        pltpu.VMEM((2, _CH), jnp.int32),
        pltpu.VMEM((2, _CH, _D), jnp.float32),
        pltpu.VMEM((2, _CH, _D), jnp.float32),
        pltpu.VMEM((2, _CH, _D), jnp.float32),
        pltpu.VMEM((_CH, _D), jnp.float32),
        pltpu.VMEM((_CH, _DH), jnp.float32),
        pltpu.VMEM_SHARED((_N, _D), jnp.float32),
        pltpu.SemaphoreType.DMA,
        pltpu.SemaphoreType.DMA,
        pltpu.SemaphoreType.DMA,
        pltpu.SemaphoreType.DMA,
    ],
    )


def _sc_score_body(src2_h, dst2_h, ts_h, efw_h, w2_h,
                   part_h,
                   ibs, ibd, g1, g2, efwv, w2v, outv, sem0, sem1):
    c = lax.axis_index("c")
    s = lax.axis_index("s")
    cE = c * _E
    base = s * _EPW
    sems = (sem0, sem1)
    pltpu.sync_copy(w2_h.at[c], w2v)

    def _issue(b, k):
        off = pl.multiple_of(base + k * _CH, 8)
        pltpu.sync_copy(src2_h.at[pl.ds(cE + off, _CH)], ibs.at[b])
        pltpu.sync_copy(dst2_h.at[pl.ds(cE + off, _CH)], ibd.at[b])
        pltpu.async_copy(ts_h.at[ibs.at[b]], g1.at[b], sems[b])
        pltpu.async_copy(ts_h.at[ibd.at[b]], g2.at[b], sems[b])
        pltpu.async_copy(efw_h.at[pl.ds(cE + off, _CH)], efwv.at[b], sems[b])

    def _wait(b, k):
        off = pl.multiple_of(base + k * _CH, 8)
        pltpu.make_async_copy(ts_h.at[ibs.at[b]], g1.at[b],
                              sems[b]).wait()
        pltpu.make_async_copy(ts_h.at[ibd.at[b]], g2.at[b],
                              sems[b]).wait()
        pltpu.make_async_copy(efw_h.at[pl.ds(cE + off, _CH)], efwv.at[b],
                              sems[b]).wait()

    _issue(0, 0)

    def outer(k2, carry):
        for b in range(2):
            k = k2 * 2 + b

            @pl.when(k + 1 < _NCH)
            def _prefetch():
                _issue(1 - b, k + 1)

            _wait(b, k)

            def row(r, carry2):
                acc = jnp.zeros((16,), jnp.float32)
                for jj in range(_DH // 16):
                    sl = pl.ds(jj * 16, 16)
                    z = (g1[b, r, sl] + g2[b, r, pl.ds(_DH + jj * 16, 16)]
                         + efwv[b, r, sl])
                    acc = acc + jnp.maximum(z, 0.0) * w2v[0, sl]
                outv[r] = acc
                return carry2

            lax.fori_loop(0, _CH, row, 0)
            off = pl.multiple_of(base + k * _CH, 8)
            pltpu.sync_copy(outv, part_h.at[pl.ds(cE + off, _CH)])
        return carry

    lax.fori_loop(0, _NCH // 2, outer, 0)


@functools.cache
def _sc_score():
    return pl.kernel(
    _sc_score_body,
    out_type=jax.ShapeDtypeStruct((2 * _E, 16), jnp.float32),
    mesh=plsc.VectorSubcoreMesh(core_axis_name="c", subcore_axis_name="s",
                                num_cores=2, num_subcores=_NS),
    compiler_params=pltpu.CompilerParams(needs_layout_passes=False),
    scratch_types=[
        pltpu.VMEM((2, _CH), jnp.int32),
        pltpu.VMEM((2, _CH), jnp.int32),
        pltpu.VMEM((2, _CH, _D), jnp.float32),
        pltpu.VMEM((2, _CH, _D), jnp.float32),
        pltpu.VMEM((2, _CH, _DH), jnp.float32),
        pltpu.VMEM((1, _DH), jnp.float32),
        pltpu.VMEM((_CH, 16), jnp.float32),
        pltpu.SemaphoreType.DMA,
        pltpu.SemaphoreType.DMA,
    ],
    )


def kernel(x, edge_index, e, Wn, bn, We, be, A, B, C, U, V, eb, nb,
           W1, b1, W2, b2):
    src = edge_index[0]
    dst = edge_index[1]
    src2 = jnp.concatenate([src, src + _N])
    dst2 = jnp.concatenate([dst, dst + _N])
    zeros = jnp.zeros((_N, _D), jnp.float32)
    h = _node_encode(x, Wn, bn)
    ef = _edge_encode(e, We, be)
    for l in range(A.shape[0]):
        tsrc, tdst = _node_mats(h, A[l], V[l], B[l])
        efx = _edge_mat_packed(ef, C[l], eb[l])
        efo, acc = _sc_edge()(dst, src2, dst2, zeros, tsrc, tdst,
                              efx.reshape(2 * _E, _D))
        ef = efo.reshape(2, _E, _DH)
        h = _node_update(h, U[l], nb[l], acc.reshape(2, _N, _D))
    ts = _tables2(h, W1[:_D], W1[_D:2 * _D])
    efw = _edge_mat(ef, W1[2 * _D:], b1)
    part = _sc_score()(src2, dst2, ts, efw.reshape(2 * _E, _DH),
                       W2[:, 0].reshape(2, 1, _DH))
    comb = _combine(part.reshape(2, _E, 16), b2.reshape(1, 1))
    return comb.reshape(_E)


# speedup vs baseline: 2.3918x; 1.0001x over previous
import functools

import jax
import jax.numpy as jnp
from jax import lax
from jax.experimental import pallas as pl
from jax.experimental.pallas import tpu as pltpu
from jax.experimental.pallas import tpu_sc as plsc

_N = 10000
_E = 320000
_D = 128
_DH = 64
_NS = 16
_CH = 40
_EPW = _E // _NS
_NCH = _EPW // _CH
_ZR = (_N // _NS) // 8 * 8
_ZREM = _N - _NS * _ZR

_BN = 2000
_BE = 2000


def _mm_kernel(x_ref, w_ref, b_ref, o_ref):
    o_ref[...] = (
        jnp.dot(x_ref[...], w_ref[...], preferred_element_type=jnp.float32)
        + b_ref[...]
    )


def _node_encode(x, Wn, bn):
    return pl.pallas_call(
        _mm_kernel,
        grid=(_N // _BN,),
        in_specs=[
            pl.BlockSpec((_BN, _D), lambda i: (i, 0)),
            pl.BlockSpec((_D, _D), lambda i: (0, 0)),
            pl.BlockSpec((1, _D), lambda i: (0, 0)),
        ],
        out_specs=pl.BlockSpec((_BN, _D), lambda i: (i, 0)),
        out_shape=jax.ShapeDtypeStruct((_N, _D), jnp.float32),
    )(x, Wn, bn.reshape(1, _D))


def _split_mm_kernel(x_ref, w_ref, b_ref, o_ref):
    y = (
        jnp.dot(x_ref[...], w_ref[...], preferred_element_type=jnp.float32)
        + b_ref[...]
    )
    o_ref[0] = y[:, :_DH]
    o_ref[1] = y[:, _DH:]


def _edge_encode(e, We, be):
    de = e.shape[1]
    return pl.pallas_call(
        _split_mm_kernel,
        grid=(_E // _BE,),
        in_specs=[
            pl.BlockSpec((_BE, de), lambda i: (i, 0)),
            pl.BlockSpec((de, _D), lambda i: (0, 0)),
            pl.BlockSpec((1, _D), lambda i: (0, 0)),
        ],
        out_specs=pl.BlockSpec((2, _BE, _DH), lambda i: (0, i, 0)),
        out_shape=jax.ShapeDtypeStruct((2, _E, _DH), jnp.float32),
    )(e, We, be.reshape(1, _D))


def _edge_mat_kernel(ef_ref, w_ref, b_ref, o_ref):
    xx = jnp.concatenate([ef_ref[0], ef_ref[1]], axis=1)
    y = jnp.dot(xx, w_ref[...], preferred_element_type=jnp.float32) + b_ref[...]
    o_ref[0] = y[:, :_DH]
    o_ref[1] = y[:, _DH:]


def _edge_mat(ef2, W, b):
    return pl.pallas_call(
        _edge_mat_kernel,
        grid=(_E // _BE,),
        in_specs=[
            pl.BlockSpec((2, _BE, _DH), lambda i: (0, i, 0)),
            pl.BlockSpec((_D, _D), lambda i: (0, 0)),
            pl.BlockSpec((1, _D), lambda i: (0, 0)),
        ],
        out_specs=pl.BlockSpec((2, _BE, _DH), lambda i: (0, i, 0)),
        out_shape=jax.ShapeDtypeStruct((2, _E, _DH), jnp.float32),
    )(ef2, W, b.reshape(1, _D))


def _edge_mat_packed_kernel(ef_ref, w_ref, b_ref, o_ref):
    xx = jnp.concatenate([ef_ref[0], ef_ref[1]], axis=1)
    y = jnp.dot(xx, w_ref[...], preferred_element_type=jnp.float32) + b_ref[...]
    o_ref[0] = jnp.concatenate([ef_ref[0], y[:, :_DH]], axis=1)
    o_ref[1] = jnp.concatenate([ef_ref[1], y[:, _DH:]], axis=1)


def _edge_mat_packed(ef2, W, b):
    return pl.pallas_call(
        _edge_mat_packed_kernel,
        grid=(_E // _BE,),
        in_specs=[
            pl.BlockSpec((2, _BE, _DH), lambda i: (0, i, 0)),
            pl.BlockSpec((_D, _D), lambda i: (0, 0)),
            pl.BlockSpec((1, _D), lambda i: (0, 0)),
        ],
        out_specs=pl.BlockSpec((2, _BE, _D), lambda i: (0, i, 0)),
        out_shape=jax.ShapeDtypeStruct((2, _E, _D), jnp.float32),
    )(ef2, W, b.reshape(1, _D))


def _node_mats_kernel(h_ref, a_ref, v_ref, bc_ref, bo_ref, ts_ref, td_ref):
    h = h_ref[...]
    ha = jnp.dot(h, a_ref[0], preferred_element_type=jnp.float32)
    hv = jnp.dot(h, v_ref[0], preferred_element_type=jnp.float32)
    ts_ref[...] = jnp.concatenate([ha, hv], axis=1)
    hbc = jnp.dot(h, bc_ref[0], preferred_element_type=jnp.float32)
    hbo = jnp.dot(h, bo_ref[0], preferred_element_type=jnp.float32)
    td_ref[...] = jnp.concatenate([hbc, hbo], axis=1)


def _halves(W):
    return W.reshape(_D, 2, _DH).transpose(1, 0, 2)


def _node_mats(h, Al, Vl, Bl):
    nb = _N // _BN
    return pl.pallas_call(
        _node_mats_kernel,
        grid=(2, nb),
        in_specs=[
            pl.BlockSpec((_BN, _D), lambda c, i: (i, 0)),
            pl.BlockSpec((1, _D, _DH), lambda c, i: (c, 0, 0)),
            pl.BlockSpec((1, _D, _DH), lambda c, i: (c, 0, 0)),
            pl.BlockSpec((1, _D, _DH), lambda c, i: (c, 0, 0)),
            pl.BlockSpec((1, _D, _DH), lambda c, i: (1 - c, 0, 0)),
        ],
        out_specs=[
            pl.BlockSpec((_BN, _D), lambda c, i: (c * nb + i, 0)),
            pl.BlockSpec((_BN, _D), lambda c, i: (c * nb + i, 0)),
        ],
        out_shape=[
            jax.ShapeDtypeStruct((2 * _N, _D), jnp.float32),
            jax.ShapeDtypeStruct((2 * _N, _D), jnp.float32),
        ],
    )(h, _halves(Al), _halves(Vl), _halves(Bl), _halves(Bl))


def _tables2_kernel(h_ref, wa_ref, wb_ref, o_ref):
    h = h_ref[...]
    hs = jnp.dot(h, wa_ref[0], preferred_element_type=jnp.float32)
    hd = jnp.dot(h, wb_ref[0], preferred_element_type=jnp.float32)
    o_ref[...] = jnp.concatenate([hs, hd], axis=1)


def _tables2(h, Wa, Wb):
    nb = _N // _BN
    return pl.pallas_call(
        _tables2_kernel,
        grid=(2, nb),
        in_specs=[
            pl.BlockSpec((_BN, _D), lambda c, i: (i, 0)),
            pl.BlockSpec((1, _D, _DH), lambda c, i: (c, 0, 0)),
            pl.BlockSpec((1, _D, _DH), lambda c, i: (c, 0, 0)),
        ],
        out_specs=pl.BlockSpec((_BN, _D), lambda c, i: (c * nb + i, 0)),
        out_shape=jax.ShapeDtypeStruct((2 * _N, _D), jnp.float32),
    )(h, _halves(Wa), _halves(Wb))


def _node_update_kernel(h_ref, u_ref, nb_ref, acc_ref, o_ref):
    h = h_ref[...]
    a0 = acc_ref[0]
    a1 = acc_ref[1]
    agg = jnp.concatenate([a0[:, :_DH], a1[:, :_DH]], axis=1)
    den = jnp.concatenate([a0[:, _DH:], a1[:, _DH:]], axis=1) + 1e-6
    hu = jnp.dot(h, u_ref[...], preferred_element_type=jnp.float32)
    o_ref[...] = h + jnp.maximum(hu + nb_ref[...] + agg / den, 0.0)


def _node_update(h, Ul, nbl, acc3):
    return pl.pallas_call(
        _node_update_kernel,
        grid=(_N // _BN,),
        in_specs=[
            pl.BlockSpec((_BN, _D), lambda i: (i, 0)),
            pl.BlockSpec((_D, _D), lambda i: (0, 0)),
            pl.BlockSpec((1, _D), lambda i: (0, 0)),
            pl.BlockSpec((2, _BN, _D), lambda i: (0, i, 0)),
        ],
        out_specs=pl.BlockSpec((_BN, _D), lambda i: (i, 0)),
        out_shape=jax.ShapeDtypeStruct((_N, _D), jnp.float32),
    )(h, Ul, nbl.reshape(1, _D), acc3)


def _combine_kernel(p_ref, b_ref, o_ref):
    s = p_ref[0] + p_ref[1]
    o_ref[...] = jnp.sum(s, axis=1, keepdims=True) + b_ref[...]


def _combine(part3, b2sc):
    return pl.pallas_call(
        _combine_kernel,
        grid=(_E // _BE,),
        in_specs=[
            pl.BlockSpec((2, _BE, 16), lambda i: (0, i, 0)),
            pl.BlockSpec((1, 1), lambda i: (0, 0)),
        ],
        out_specs=pl.BlockSpec((_BE, 1), lambda i: (i, 0)),
        out_shape=jax.ShapeDtypeStruct((_E, 1), jnp.float32),
    )(part3, b2sc)


def _sc_edge_body(dst_h, src2_h, dst2_h, zeros_h, tsrc_h, tdst_h, efx_h,
                  efo_h, acc_h,
                  is0, is1, id0, id1, ir0, ir1, gsrc, gdst, efx, scat, efov,
                  accsh, sem0, sem1, sem_eo, sem_sc):
    c = lax.axis_index("c")
    s = lax.axis_index("s")
    cN = c * _N
    cE = c * _E
    base = s * _EPW
    sems = (sem0, sem1)
    isb = (is0, is1)
    idb = (id0, id1)
    irb = (ir0, ir1)

    pltpu.sync_copy(zeros_h.at[pl.ds(s * _ZR, _ZR)],
                    accsh.at[pl.ds(s * _ZR, _ZR)])

    @pl.when(s == 0)
    def _zero_rem():
        pltpu.sync_copy(zeros_h.at[pl.ds(_NS * _ZR, _ZREM)],
                        accsh.at[pl.ds(_NS * _ZR, _ZREM)])

    plsc.subcore_barrier()

    def _issue(b, k):
        off = pl.multiple_of(base + k * _CH, 8)
        pltpu.sync_copy(src2_h.at[pl.ds(cE + off, _CH)], isb[b])
        pltpu.sync_copy(dst2_h.at[pl.ds(cE + off, _CH)], idb[b])
        pltpu.sync_copy(dst_h.at[pl.ds(off, _CH)], irb[b])
        pltpu.async_copy(tsrc_h.at[isb[b]], gsrc.at[b], sems[b])
        pltpu.async_copy(tdst_h.at[idb[b]], gdst.at[b], sems[b])
        pltpu.async_copy(efx_h.at[pl.ds(cE + off, _CH)], efx.at[b], sems[b])

    def _wait(b, k):
        off = pl.multiple_of(base + k * _CH, 8)
        pltpu.make_async_copy(tsrc_h.at[isb[b]], gsrc.at[b], sems[b]).wait()
        pltpu.make_async_copy(tdst_h.at[idb[b]], gdst.at[b], sems[b]).wait()
        pltpu.make_async_copy(efx_h.at[pl.ds(cE + off, _CH)], efx.at[b],
                              sems[b]).wait()

    def _wait_out(kp):
        offp = pl.multiple_of(base + kp * _CH, 8)
        pltpu.make_async_copy(efov, efo_h.at[pl.ds(cE + offp, _CH)],
                              sem_eo).wait()

    def _compute(b, k):
        _wait(b, k)

        @pl.when(k > 0)
        def _dr():
            _wait_out(k - 1)

        def row(r, carry2):
            for jj in range(_DH // 16):
                sl = pl.ds(jj * 16, 16)
                slv = pl.ds(_DH + jj * 16, 16)
                a_ = gsrc[b, r, sl]
                v_ = gsrc[b, r, slv]
                b_ = gdst[b, r, sl]
                f_ = efx[b, r, sl]
                t_ = efx[b, r, slv]
                fn = f_ + jnp.maximum(a_ + b_ + t_, 0.0)
                efov[r, sl] = fn
                eta = 1.0 / (1.0 + jnp.exp(-fn))
                scat[r, slv] = eta
                scat[r, sl] = eta * v_
            return carry2

        lax.fori_loop(0, _CH, row, 0)
        off = pl.multiple_of(base + k * _CH, 8)
        pltpu.async_copy(efov, efo_h.at[pl.ds(cE + off, _CH)], sem_eo)
        pltpu.sync_copy(scat, accsh.at[irb[b]], add=True)

    _issue(0, 0)

    def outer(t, carry):
        k = 2 * t
        _issue(1, k + 1)
        _compute(0, k)

        @pl.when(t + 1 < _NCH // 2)
        def _pf():
            _issue(0, k + 2)

        _compute(1, k + 1)
        return carry

    lax.fori_loop(0, _NCH // 2, outer, 0)
    _wait_out(_NCH - 1)
    plsc.subcore_barrier()
    pltpu.sync_copy(accsh.at[pl.ds(s * _ZR, _ZR)],
                    acc_h.at[pl.ds(cN + s * _ZR, _ZR)])

    @pl.when(s == 0)
    def _wb_rem():
        pltpu.sync_copy(accsh.at[pl.ds(_NS * _ZR, _ZREM)],
                        acc_h.at[pl.ds(cN + _NS * _ZR, _ZREM)])


@functools.cache
def _sc_edge():
    return pl.kernel(
    _sc_edge_body,
    out_type=[
        jax.ShapeDtypeStruct((2 * _E, _DH), jnp.float32),
        jax.ShapeDtypeStruct((2 * _N, _D), jnp.float32),
    ],
    mesh=plsc.VectorSubcoreMesh(core_axis_name="c", subcore_axis_name="s",
                                num_cores=2, num_subcores=_NS),
    compiler_params=pltpu.CompilerParams(needs_layout_passes=False),
    scratch_types=[
        pltpu.VMEM((_CH,), jnp.int32),
        pltpu.VMEM((_CH,), jnp.int32),
        pltpu.VMEM((_CH,), jnp.int32),
        pltpu.VMEM((_CH,), jnp.int32),
        pltpu.VMEM((_CH,), jnp.int32),
        pltpu.VMEM((_CH,), jnp.int32),
        pltpu.VMEM((2, _CH, _D), jnp.float32),
        pltpu.VMEM((2, _CH, _D), jnp.float32),
        pltpu.VMEM((2, _CH, _D), jnp.float32),
        pltpu.VMEM((_CH, _D), jnp.float32),
        pltpu.VMEM((_CH, _DH), jnp.float32),
        pltpu.VMEM_SHARED((_N, _D), jnp.float32),
        pltpu.SemaphoreType.DMA,
        pltpu.SemaphoreType.DMA,
        pltpu.SemaphoreType.DMA,
        pltpu.SemaphoreType.DMA,
    ],
    )


def _sc_score_body(src2_h, dst2_h, ts_h, efw_h, w2_h,
                   part_h,
                   is0, is1, id0, id1, g1, g2, efwv, w2v, outv, sem0, sem1):
    c = lax.axis_index("c")
    s = lax.axis_index("s")
    cE = c * _E
    base = s * _EPW
    sems = (sem0, sem1)
    isb = (is0, is1)
    idb = (id0, id1)
    pltpu.sync_copy(w2_h.at[c], w2v)

    def _issue(b, k):
        off = pl.multiple_of(base + k * _CH, 8)
        pltpu.sync_copy(src2_h.at[pl.ds(cE + off, _CH)], isb[b])
        pltpu.sync_copy(dst2_h.at[pl.ds(cE + off, _CH)], idb[b])
        pltpu.async_copy(ts_h.at[isb[b]], g1.at[b], sems[b])
        pltpu.async_copy(ts_h.at[idb[b]], g2.at[b], sems[b])
        pltpu.async_copy(efw_h.at[pl.ds(cE + off, _CH)], efwv.at[b], sems[b])

    def _wait(b, k):
        off = pl.multiple_of(base + k * _CH, 8)
        pltpu.make_async_copy(ts_h.at[isb[b]], g1.at[b], sems[b]).wait()
        pltpu.make_async_copy(ts_h.at[idb[b]], g2.at[b], sems[b]).wait()
        pltpu.make_async_copy(efw_h.at[pl.ds(cE + off, _CH)], efwv.at[b],
                              sems[b]).wait()

    def _compute(b, k):
        _wait(b, k)

        def row(r, carry2):
            acc = jnp.zeros((16,), jnp.float32)
            for jj in range(_DH // 16):
                sl = pl.ds(jj * 16, 16)
                z = (g1[b, r, sl] + g2[b, r, pl.ds(_DH + jj * 16, 16)]
                     + efwv[b, r, sl])
                acc = acc + jnp.maximum(z, 0.0) * w2v[0, sl]
            outv[r] = acc
            return carry2

        lax.fori_loop(0, _CH, row, 0)
        off = pl.multiple_of(base + k * _CH, 8)
        pltpu.sync_copy(outv, part_h.at[pl.ds(cE + off, _CH)])

    _issue(0, 0)

    def outer(t, carry):
        k = 2 * t
        _issue(1, k + 1)
        _compute(0, k)

        @pl.when(t + 1 < _NCH // 2)
        def _pf():
            _issue(0, k + 2)

        _compute(1, k + 1)
        return carry

    lax.fori_loop(0, _NCH // 2, outer, 0)


@functools.cache
def _sc_score():
    return pl.kernel(
    _sc_score_body,
    out_type=jax.ShapeDtypeStruct((2 * _E, 16), jnp.float32),
    mesh=plsc.VectorSubcoreMesh(core_axis_name="c", subcore_axis_name="s",
                                num_cores=2, num_subcores=_NS),
    compiler_params=pltpu.CompilerParams(needs_layout_passes=False),
    scratch_types=[
        pltpu.VMEM((_CH,), jnp.int32),
        pltpu.VMEM((_CH,), jnp.int32),
        pltpu.VMEM((_CH,), jnp.int32),
        pltpu.VMEM((_CH,), jnp.int32),
        pltpu.VMEM((2, _CH, _D), jnp.float32),
        pltpu.VMEM((2, _CH, _D), jnp.float32),
        pltpu.VMEM((2, _CH, _DH), jnp.float32),
        pltpu.VMEM((1, _DH), jnp.float32),
        pltpu.VMEM((_CH, 16), jnp.float32),
        pltpu.SemaphoreType.DMA,
        pltpu.SemaphoreType.DMA,
    ],
    )


def kernel(x, edge_index, e, Wn, bn, We, be, A, B, C, U, V, eb, nb,
           W1, b1, W2, b2):
    src = edge_index[0]
    dst = edge_index[1]
    src2 = jnp.concatenate([src, src + _N])
    dst2 = jnp.concatenate([dst, dst + _N])
    zeros = jnp.zeros((_N, _D), jnp.float32)
    h = _node_encode(x, Wn, bn)
    ef = _edge_encode(e, We, be)
    for l in range(A.shape[0]):
        tsrc, tdst = _node_mats(h, A[l], V[l], B[l])
        efx = _edge_mat_packed(ef, C[l], eb[l])
        efo, acc = _sc_edge()(dst, src2, dst2, zeros, tsrc, tdst,
                              efx.reshape(2 * _E, _D))
        ef = efo.reshape(2, _E, _DH)
        h = _node_update(h, U[l], nb[l], acc.reshape(2, _N, _D))
    ts = _tables2(h, W1[:_D], W1[_D:2 * _D])
    efw = _edge_mat(ef, W1[2 * _D:], b1)
    part = _sc_score()(src2, dst2, ts, efw.reshape(2 * _E, _DH),
                       W2[:, 0].reshape(2, 1, _DH))
    comb = _combine(part.reshape(2, _E, 16), b2.reshape(1, 1))
    return comb.reshape(_E)


# speedup vs baseline: 3.2685x; 1.3665x over previous
import functools

import jax
import jax.numpy as jnp
from jax import lax
from jax.experimental import pallas as pl
from jax.experimental.pallas import tpu as pltpu
from jax.experimental.pallas import tpu_sc as plsc

_N = 10000
_E = 320000
_D = 128
_DH = 64
_NS = 16
_CH = 40
_EPW = _E // _NS
_NCH = _EPW // _CH
_ZR = (_N // _NS) // 8 * 8
_ZREM = _N - _NS * _ZR

_BN = 2000
_BE = 2000


def _mm_kernel(x_ref, w_ref, b_ref, o_ref):
    o_ref[...] = (
        jnp.dot(x_ref[...], w_ref[...], preferred_element_type=jnp.float32)
        + b_ref[...]
    )


def _node_encode(x, Wn, bn):
    return pl.pallas_call(
        _mm_kernel,
        grid=(_N // _BN,),
        in_specs=[
            pl.BlockSpec((_BN, _D), lambda i: (i, 0)),
            pl.BlockSpec((_D, _D), lambda i: (0, 0)),
            pl.BlockSpec((1, _D), lambda i: (0, 0)),
        ],
        out_specs=pl.BlockSpec((_BN, _D), lambda i: (i, 0)),
        out_shape=jax.ShapeDtypeStruct((_N, _D), jnp.float32),
    )(x, Wn, bn.reshape(1, _D))


def _split_mm_kernel(x_ref, w_ref, b_ref, o_ref):
    y = (
        jnp.dot(x_ref[...], w_ref[...], preferred_element_type=jnp.float32)
        + b_ref[...]
    )
    o_ref[0] = y[:, :_DH]
    o_ref[1] = y[:, _DH:]


def _edge_encode(e, We, be):
    de = e.shape[1]
    return pl.pallas_call(
        _split_mm_kernel,
        grid=(_E // _BE,),
        in_specs=[
            pl.BlockSpec((_BE, de), lambda i: (i, 0)),
            pl.BlockSpec((de, _D), lambda i: (0, 0)),
            pl.BlockSpec((1, _D), lambda i: (0, 0)),
        ],
        out_specs=pl.BlockSpec((2, _BE, _DH), lambda i: (0, i, 0)),
        out_shape=jax.ShapeDtypeStruct((2, _E, _DH), jnp.float32),
    )(e, We, be.reshape(1, _D))


def _edge_mat_kernel(ef_ref, w_ref, b_ref, o_ref):
    xx = jnp.concatenate([ef_ref[0], ef_ref[1]], axis=1)
    y = jnp.dot(xx, w_ref[...], preferred_element_type=jnp.float32) + b_ref[...]
    o_ref[0] = y[:, :_DH]
    o_ref[1] = y[:, _DH:]


def _edge_mat(ef2, W, b):
    return pl.pallas_call(
        _edge_mat_kernel,
        grid=(_E // _BE,),
        in_specs=[
            pl.BlockSpec((2, _BE, _DH), lambda i: (0, i, 0)),
            pl.BlockSpec((_D, _D), lambda i: (0, 0)),
            pl.BlockSpec((1, _D), lambda i: (0, 0)),
        ],
        out_specs=pl.BlockSpec((2, _BE, _DH), lambda i: (0, i, 0)),
        out_shape=jax.ShapeDtypeStruct((2, _E, _DH), jnp.float32),
    )(ef2, W, b.reshape(1, _D))


def _edge_mat_packed_kernel(ef_ref, w_ref, b_ref, o_ref):
    xx = jnp.concatenate([ef_ref[0], ef_ref[1]], axis=1)
    y = jnp.dot(xx, w_ref[...], preferred_element_type=jnp.float32) + b_ref[...]
    o_ref[0] = jnp.concatenate([ef_ref[0], y[:, :_DH]], axis=1)
    o_ref[1] = jnp.concatenate([ef_ref[1], y[:, _DH:]], axis=1)


def _edge_mat_packed(ef2, W, b):
    return pl.pallas_call(
        _edge_mat_packed_kernel,
        grid=(_E // _BE,),
        in_specs=[
            pl.BlockSpec((2, _BE, _DH), lambda i: (0, i, 0)),
            pl.BlockSpec((_D, _D), lambda i: (0, 0)),
            pl.BlockSpec((1, _D), lambda i: (0, 0)),
        ],
        out_specs=pl.BlockSpec((2, _BE, _D), lambda i: (0, i, 0)),
        out_shape=jax.ShapeDtypeStruct((2, _E, _D), jnp.float32),
    )(ef2, W, b.reshape(1, _D))


def _node_mats_kernel(h_ref, a_ref, v_ref, bc_ref, bo_ref, ts_ref, td_ref):
    h = h_ref[...]
    ha = jnp.dot(h, a_ref[0], preferred_element_type=jnp.float32)
    hv = jnp.dot(h, v_ref[0], preferred_element_type=jnp.float32)
    ts_ref[...] = jnp.concatenate([ha, hv], axis=1)
    hbc = jnp.dot(h, bc_ref[0], preferred_element_type=jnp.float32)
    hbo = jnp.dot(h, bo_ref[0], preferred_element_type=jnp.float32)
    td_ref[...] = jnp.concatenate([hbc, hbo], axis=1)


def _halves(W):
    return W.reshape(_D, 2, _DH).transpose(1, 0, 2)


def _node_mats(h, Al, Vl, Bl):
    nb = _N // _BN
    return pl.pallas_call(
        _node_mats_kernel,
        grid=(2, nb),
        in_specs=[
            pl.BlockSpec((_BN, _D), lambda c, i: (i, 0)),
            pl.BlockSpec((1, _D, _DH), lambda c, i: (c, 0, 0)),
            pl.BlockSpec((1, _D, _DH), lambda c, i: (c, 0, 0)),
            pl.BlockSpec((1, _D, _DH), lambda c, i: (c, 0, 0)),
            pl.BlockSpec((1, _D, _DH), lambda c, i: (1 - c, 0, 0)),
        ],
        out_specs=[
            pl.BlockSpec((_BN, _D), lambda c, i: (c * nb + i, 0)),
            pl.BlockSpec((_BN, _D), lambda c, i: (c * nb + i, 0)),
        ],
        out_shape=[
            jax.ShapeDtypeStruct((2 * _N, _D), jnp.float32),
            jax.ShapeDtypeStruct((2 * _N, _D), jnp.float32),
        ],
    )(h, _halves(Al), _halves(Vl), _halves(Bl), _halves(Bl))


def _tables2_kernel(h_ref, wa_ref, wb_ref, o_ref):
    h = h_ref[...]
    hs = jnp.dot(h, wa_ref[0], preferred_element_type=jnp.float32)
    hd = jnp.dot(h, wb_ref[0], preferred_element_type=jnp.float32)
    o_ref[...] = jnp.concatenate([hs, hd], axis=1)


def _tables2(h, Wa, Wb):
    nb = _N // _BN
    return pl.pallas_call(
        _tables2_kernel,
        grid=(2, nb),
        in_specs=[
            pl.BlockSpec((_BN, _D), lambda c, i: (i, 0)),
            pl.BlockSpec((1, _D, _DH), lambda c, i: (c, 0, 0)),
            pl.BlockSpec((1, _D, _DH), lambda c, i: (c, 0, 0)),
        ],
        out_specs=pl.BlockSpec((_BN, _D), lambda c, i: (c * nb + i, 0)),
        out_shape=jax.ShapeDtypeStruct((2 * _N, _D), jnp.float32),
    )(h, _halves(Wa), _halves(Wb))


def _node_update_kernel(h_ref, u_ref, nb_ref, acc_ref, o_ref):
    h = h_ref[...]
    a0 = acc_ref[0]
    a1 = acc_ref[1]
    agg = jnp.concatenate([a0[:, :_DH], a1[:, :_DH]], axis=1)
    den = jnp.concatenate([a0[:, _DH:], a1[:, _DH:]], axis=1) + 1e-6
    hu = jnp.dot(h, u_ref[...], preferred_element_type=jnp.float32)
    o_ref[...] = h + jnp.maximum(hu + nb_ref[...] + agg / den, 0.0)


def _node_update(h, Ul, nbl, acc3):
    return pl.pallas_call(
        _node_update_kernel,
        grid=(_N // _BN,),
        in_specs=[
            pl.BlockSpec((_BN, _D), lambda i: (i, 0)),
            pl.BlockSpec((_D, _D), lambda i: (0, 0)),
            pl.BlockSpec((1, _D), lambda i: (0, 0)),
            pl.BlockSpec((2, _BN, _D), lambda i: (0, i, 0)),
        ],
        out_specs=pl.BlockSpec((_BN, _D), lambda i: (i, 0)),
        out_shape=jax.ShapeDtypeStruct((_N, _D), jnp.float32),
    )(h, Ul, nbl.reshape(1, _D), acc3)


def _combine_kernel(p_ref, b_ref, o_ref):
    s = p_ref[0] + p_ref[1]
    o_ref[...] = jnp.sum(s, axis=1, keepdims=True) + b_ref[...]


def _combine(part3, b2sc):
    return pl.pallas_call(
        _combine_kernel,
        grid=(_E // _BE,),
        in_specs=[
            pl.BlockSpec((2, _BE, 16), lambda i: (0, i, 0)),
            pl.BlockSpec((1, 1), lambda i: (0, 0)),
        ],
        out_specs=pl.BlockSpec((_BE, 1), lambda i: (i, 0)),
        out_shape=jax.ShapeDtypeStruct((_E, 1), jnp.float32),
    )(part3, b2sc)


def _sc_edge_body(dst_h, src2_h, dst2_h, zeros_h, tsrc_h, tdst_h, efx_h,
                  efo_h, acc_h,
                  is0, is1, id0, id1, ir0, ir1, gsrc, gdst, efx, scat, efov,
                  accsh, sem0, sem1, sem_eo, sem_ix):
    c = lax.axis_index("c")
    s = lax.axis_index("s")
    cN = c * _N
    cE = c * _E
    base = s * _EPW
    sems = (sem0, sem1)
    isb = (is0, is1)
    idb = (id0, id1)
    irb = (ir0, ir1)

    pltpu.sync_copy(zeros_h.at[pl.ds(s * _ZR, _ZR)],
                    accsh.at[pl.ds(s * _ZR, _ZR)])

    @pl.when(s == 0)
    def _zero_rem():
        pltpu.sync_copy(zeros_h.at[pl.ds(_NS * _ZR, _ZREM)],
                        accsh.at[pl.ds(_NS * _ZR, _ZREM)])

    plsc.subcore_barrier()

    def _idx_issue(b, k):
        off = pl.multiple_of(base + k * _CH, 8)
        pltpu.async_copy(src2_h.at[pl.ds(cE + off, _CH)], isb[b], sem_ix)
        pltpu.async_copy(dst2_h.at[pl.ds(cE + off, _CH)], idb[b], sem_ix)

    def _issue(b, k):
        off = pl.multiple_of(base + k * _CH, 8)
        pltpu.make_async_copy(src2_h.at[pl.ds(cE + off, _CH)], isb[b],
                              sem_ix).wait()
        pltpu.make_async_copy(dst2_h.at[pl.ds(cE + off, _CH)], idb[b],
                              sem_ix).wait()
        pltpu.async_copy(tsrc_h.at[isb[b]], gsrc.at[b], sems[b])
        pltpu.async_copy(tdst_h.at[idb[b]], gdst.at[b], sems[b])
        pltpu.async_copy(efx_h.at[pl.ds(cE + off, _CH)], efx.at[b], sems[b])
        pltpu.async_copy(dst_h.at[pl.ds(off, _CH)], irb[b], sems[b])

    def _wait(b, k):
        off = pl.multiple_of(base + k * _CH, 8)
        pltpu.make_async_copy(tsrc_h.at[isb[b]], gsrc.at[b], sems[b]).wait()
        pltpu.make_async_copy(tdst_h.at[idb[b]], gdst.at[b], sems[b]).wait()
        pltpu.make_async_copy(efx_h.at[pl.ds(cE + off, _CH)], efx.at[b],
                              sems[b]).wait()
        pltpu.make_async_copy(dst_h.at[pl.ds(off, _CH)], irb[b],
                              sems[b]).wait()

    def _wait_out(kp):
        offp = pl.multiple_of(base + kp * _CH, 8)
        pltpu.make_async_copy(efov, efo_h.at[pl.ds(cE + offp, _CH)],
                              sem_eo).wait()

    def _compute(b, k):
        @pl.when(k > 0)
        def _dr():
            _wait_out(k - 1)

        def row(r, carry2):
            for jj in range(_DH // 16):
                sl = pl.ds(jj * 16, 16)
                slv = pl.ds(_DH + jj * 16, 16)
                a_ = gsrc[b, r, sl]
                v_ = gsrc[b, r, slv]
                b_ = gdst[b, r, sl]
                f_ = efx[b, r, sl]
                t_ = efx[b, r, slv]
                fn = f_ + jnp.maximum(a_ + b_ + t_, 0.0)
                efov[r, sl] = fn
                eta = 1.0 / (1.0 + jnp.exp(-fn))
                scat[r, slv] = eta
                scat[r, sl] = eta * v_
            return carry2

        lax.fori_loop(0, _CH, row, 0)
        off = pl.multiple_of(base + k * _CH, 8)
        pltpu.async_copy(efov, efo_h.at[pl.ds(cE + off, _CH)], sem_eo)
        pltpu.sync_copy(scat, accsh.at[irb[b]], add=True)

    _idx_issue(0, 0)
    _issue(0, 0)
    _idx_issue(1, 1)
    _issue(1, 1)

    def outer(t, carry):
        k = 2 * t
        _wait(0, k)

        @pl.when(k + 2 < _NCH)
        def _pf0i():
            _idx_issue(0, k + 2)

        _compute(0, k)

        @pl.when(k + 2 < _NCH)
        def _pf0():
            _issue(0, k + 2)

        _wait(1, k + 1)

        @pl.when(k + 3 < _NCH)
        def _pf1i():
            _idx_issue(1, k + 3)

        _compute(1, k + 1)

        @pl.when(k + 3 < _NCH)
        def _pf1():
            _issue(1, k + 3)

        return carry

    lax.fori_loop(0, _NCH // 2, outer, 0)
    _wait_out(_NCH - 1)
    plsc.subcore_barrier()
    pltpu.sync_copy(accsh.at[pl.ds(s * _ZR, _ZR)],
                    acc_h.at[pl.ds(cN + s * _ZR, _ZR)])

    @pl.when(s == 0)
    def _wb_rem():
        pltpu.sync_copy(accsh.at[pl.ds(_NS * _ZR, _ZREM)],
                        acc_h.at[pl.ds(cN + _NS * _ZR, _ZREM)])


@functools.cache
def _sc_edge():
    return pl.kernel(
    _sc_edge_body,
    out_type=[
        jax.ShapeDtypeStruct((2 * _E, _DH), jnp.float32),
        jax.ShapeDtypeStruct((2 * _N, _D), jnp.float32),
    ],
    mesh=plsc.VectorSubcoreMesh(core_axis_name="c", subcore_axis_name="s",
                                num_cores=2, num_subcores=_NS),
    compiler_params=pltpu.CompilerParams(needs_layout_passes=False),
    scratch_types=[
        pltpu.VMEM((_CH,), jnp.int32),
        pltpu.VMEM((_CH,), jnp.int32),
        pltpu.VMEM((_CH,), jnp.int32),
        pltpu.VMEM((_CH,), jnp.int32),
        pltpu.VMEM((_CH,), jnp.int32),
        pltpu.VMEM((_CH,), jnp.int32),
        pltpu.VMEM((2, _CH, _D), jnp.float32),
        pltpu.VMEM((2, _CH, _D), jnp.float32),
        pltpu.VMEM((2, _CH, _D), jnp.float32),
        pltpu.VMEM((_CH, _D), jnp.float32),
        pltpu.VMEM((_CH, _DH), jnp.float32),
        pltpu.VMEM_SHARED((_N, _D), jnp.float32),
        pltpu.SemaphoreType.DMA,
        pltpu.SemaphoreType.DMA,
        pltpu.SemaphoreType.DMA,
        pltpu.SemaphoreType.DMA,
    ],
    )


def _sc_score_body(src2_h, dst2_h, ts_h, efw_h, w2_h,
                   part_h,
                   is0, is1, id0, id1, g1, g2, efwv, w2v, outv, sem0, sem1):
    c = lax.axis_index("c")
    s = lax.axis_index("s")
    cE = c * _E
    base = s * _EPW
    sems = (sem0, sem1)
    isb = (is0, is1)
    idb = (id0, id1)
    pltpu.sync_copy(w2_h.at[c], w2v)

    def _issue(b, k):
        off = pl.multiple_of(base + k * _CH, 8)
        pltpu.sync_copy(src2_h.at[pl.ds(cE + off, _CH)], isb[b])
        pltpu.sync_copy(dst2_h.at[pl.ds(cE + off, _CH)], idb[b])
        pltpu.async_copy(ts_h.at[isb[b]], g1.at[b], sems[b])
        pltpu.async_copy(ts_h.at[idb[b]], g2.at[b], sems[b])
        pltpu.async_copy(efw_h.at[pl.ds(cE + off, _CH)], efwv.at[b], sems[b])

    def _wait(b, k):
        off = pl.multiple_of(base + k * _CH, 8)
        pltpu.make_async_copy(ts_h.at[isb[b]], g1.at[b], sems[b]).wait()
        pltpu.make_async_copy(ts_h.at[idb[b]], g2.at[b], sems[b]).wait()
        pltpu.make_async_copy(efw_h.at[pl.ds(cE + off, _CH)], efwv.at[b],
                              sems[b]).wait()

    def _compute(b, k):
        _wait(b, k)

        def row(r, carry2):
            acc = jnp.zeros((16,), jnp.float32)
            for jj in range(_DH // 16):
                sl = pl.ds(jj * 16, 16)
                z = (g1[b, r, sl] + g2[b, r, pl.ds(_DH + jj * 16, 16)]
                     + efwv[b, r, sl])
                acc = acc + jnp.maximum(z, 0.0) * w2v[0, sl]
            outv[r] = acc
            return carry2

        lax.fori_loop(0, _CH, row, 0)
        off = pl.multiple_of(base + k * _CH, 8)
        pltpu.sync_copy(outv, part_h.at[pl.ds(cE + off, _CH)])

    _issue(0, 0)

    def outer(t, carry):
        k = 2 * t
        _issue(1, k + 1)
        _compute(0, k)

        @pl.when(t + 1 < _NCH // 2)
        def _pf():
            _issue(0, k + 2)

        _compute(1, k + 1)
        return carry

    lax.fori_loop(0, _NCH // 2, outer, 0)


@functools.cache
def _sc_score():
    return pl.kernel(
    _sc_score_body,
    out_type=jax.ShapeDtypeStruct((2 * _E, 16), jnp.float32),
    mesh=plsc.VectorSubcoreMesh(core_axis_name="c", subcore_axis_name="s",
                                num_cores=2, num_subcores=_NS),
    compiler_params=pltpu.CompilerParams(needs_layout_passes=False),
    scratch_types=[
        pltpu.VMEM((_CH,), jnp.int32),
        pltpu.VMEM((_CH,), jnp.int32),
        pltpu.VMEM((_CH,), jnp.int32),
        pltpu.VMEM((_CH,), jnp.int32),
        pltpu.VMEM((2, _CH, _D), jnp.float32),
        pltpu.VMEM((2, _CH, _D), jnp.float32),
        pltpu.VMEM((2, _CH, _DH), jnp.float32),
        pltpu.VMEM((1, _DH), jnp.float32),
        pltpu.VMEM((_CH, 16), jnp.float32),
        pltpu.SemaphoreType.DMA,
        pltpu.SemaphoreType.DMA,
    ],
    )


def kernel(x, edge_index, e, Wn, bn, We, be, A, B, C, U, V, eb, nb,
           W1, b1, W2, b2):
    src = edge_index[0]
    dst = edge_index[1]
    src2 = jnp.concatenate([src, src + _N])
    dst2 = jnp.concatenate([dst, dst + _N])
    zeros = jnp.zeros((_N, _D), jnp.float32)
    h = _node_encode(x, Wn, bn)
    ef = _edge_encode(e, We, be)
    for l in range(A.shape[0]):
        tsrc, tdst = _node_mats(h, A[l], V[l], B[l])
        efx = _edge_mat_packed(ef, C[l], eb[l])
        efo, acc = _sc_edge()(dst, src2, dst2, zeros, tsrc, tdst,
                              efx.reshape(2 * _E, _D))
        ef = efo.reshape(2, _E, _DH)
        h = _node_update(h, U[l], nb[l], acc.reshape(2, _N, _D))
    ts = _tables2(h, W1[:_D], W1[_D:2 * _D])
    efw = _edge_mat(ef, W1[2 * _D:], b1)
    part = _sc_score()(src2, dst2, ts, efw.reshape(2 * _E, _DH),
                       W2[:, 0].reshape(2, 1, _DH))
    comb = _combine(part.reshape(2, _E, 16), b2.reshape(1, 1))
    return comb.reshape(_E)


# speedup vs baseline: 3.4273x; 1.0486x over previous
import functools

import jax
import jax.numpy as jnp
from jax import lax
from jax.experimental import pallas as pl
from jax.experimental.pallas import tpu as pltpu
from jax.experimental.pallas import tpu_sc as plsc

_N = 10000
_E = 320000
_D = 128
_DH = 64
_NS = 16
_CH = 40
_EPW = _E // _NS
_NCH = _EPW // _CH
_ZR = (_N // _NS) // 8 * 8
_ZREM = _N - _NS * _ZR

_BN = 2000
_BE = 2000


def _mm_kernel(x_ref, w_ref, b_ref, o_ref):
    o_ref[...] = (
        jnp.dot(x_ref[...], w_ref[...], preferred_element_type=jnp.float32)
        + b_ref[...]
    )


def _node_encode(x, Wn, bn):
    return pl.pallas_call(
        _mm_kernel,
        grid=(_N // _BN,),
        in_specs=[
            pl.BlockSpec((_BN, _D), lambda i: (i, 0)),
            pl.BlockSpec((_D, _D), lambda i: (0, 0)),
            pl.BlockSpec((1, _D), lambda i: (0, 0)),
        ],
        out_specs=pl.BlockSpec((_BN, _D), lambda i: (i, 0)),
        out_shape=jax.ShapeDtypeStruct((_N, _D), jnp.float32),
    )(x, Wn, bn.reshape(1, _D))


def _split_mm_kernel(x_ref, w_ref, b_ref, o_ref):
    y = (
        jnp.dot(x_ref[...], w_ref[...], preferred_element_type=jnp.float32)
        + b_ref[...]
    )
    o_ref[0] = y[:, :_DH]
    o_ref[1] = y[:, _DH:]


def _edge_encode(e, We, be):
    de = e.shape[1]
    return pl.pallas_call(
        _split_mm_kernel,
        grid=(_E // _BE,),
        in_specs=[
            pl.BlockSpec((_BE, de), lambda i: (i, 0)),
            pl.BlockSpec((de, _D), lambda i: (0, 0)),
            pl.BlockSpec((1, _D), lambda i: (0, 0)),
        ],
        out_specs=pl.BlockSpec((2, _BE, _DH), lambda i: (0, i, 0)),
        out_shape=jax.ShapeDtypeStruct((2, _E, _DH), jnp.float32),
    )(e, We, be.reshape(1, _D))


def _edge_mat_kernel(ef_ref, w_ref, b_ref, o_ref):
    xx = jnp.concatenate([ef_ref[0], ef_ref[1]], axis=1)
    y = jnp.dot(xx, w_ref[...], preferred_element_type=jnp.float32) + b_ref[...]
    o_ref[0] = y[:, :_DH]
    o_ref[1] = y[:, _DH:]


def _edge_mat(ef2, W, b):
    return pl.pallas_call(
        _edge_mat_kernel,
        grid=(_E // _BE,),
        in_specs=[
            pl.BlockSpec((2, _BE, _DH), lambda i: (0, i, 0)),
            pl.BlockSpec((_D, _D), lambda i: (0, 0)),
            pl.BlockSpec((1, _D), lambda i: (0, 0)),
        ],
        out_specs=pl.BlockSpec((2, _BE, _DH), lambda i: (0, i, 0)),
        out_shape=jax.ShapeDtypeStruct((2, _E, _DH), jnp.float32),
    )(ef2, W, b.reshape(1, _D))


def _edge_mat_packed_kernel(ef_ref, w_ref, b_ref, o_ref):
    xx = jnp.concatenate([ef_ref[0], ef_ref[1]], axis=1)
    y = jnp.dot(xx, w_ref[...], preferred_element_type=jnp.float32) + b_ref[...]
    o_ref[0] = jnp.concatenate([ef_ref[0], y[:, :_DH]], axis=1)
    o_ref[1] = jnp.concatenate([ef_ref[1], y[:, _DH:]], axis=1)


def _edge_mat_packed(ef2, W, b):
    return pl.pallas_call(
        _edge_mat_packed_kernel,
        grid=(_E // _BE,),
        in_specs=[
            pl.BlockSpec((2, _BE, _DH), lambda i: (0, i, 0)),
            pl.BlockSpec((_D, _D), lambda i: (0, 0)),
            pl.BlockSpec((1, _D), lambda i: (0, 0)),
        ],
        out_specs=pl.BlockSpec((2, _BE, _D), lambda i: (0, i, 0)),
        out_shape=jax.ShapeDtypeStruct((2, _E, _D), jnp.float32),
    )(ef2, W, b.reshape(1, _D))


def _node_mats_kernel(h_ref, a_ref, v_ref, bc_ref, bo_ref, ts_ref, td_ref):
    h = h_ref[...]
    ha = jnp.dot(h, a_ref[0], preferred_element_type=jnp.float32)
    hv = jnp.dot(h, v_ref[0], preferred_element_type=jnp.float32)
    ts_ref[...] = jnp.concatenate([ha, hv], axis=1)
    hbc = jnp.dot(h, bc_ref[0], preferred_element_type=jnp.float32)
    hbo = jnp.dot(h, bo_ref[0], preferred_element_type=jnp.float32)
    td_ref[...] = jnp.concatenate([hbc, hbo], axis=1)


def _halves(W):
    return W.reshape(_D, 2, _DH).transpose(1, 0, 2)


def _node_mats(h, Al, Vl, Bl):
    nb = _N // _BN
    return pl.pallas_call(
        _node_mats_kernel,
        grid=(2, nb),
        in_specs=[
            pl.BlockSpec((_BN, _D), lambda c, i: (i, 0)),
            pl.BlockSpec((1, _D, _DH), lambda c, i: (c, 0, 0)),
            pl.BlockSpec((1, _D, _DH), lambda c, i: (c, 0, 0)),
            pl.BlockSpec((1, _D, _DH), lambda c, i: (c, 0, 0)),
            pl.BlockSpec((1, _D, _DH), lambda c, i: (1 - c, 0, 0)),
        ],
        out_specs=[
            pl.BlockSpec((_BN, _D), lambda c, i: (c * nb + i, 0)),
            pl.BlockSpec((_BN, _D), lambda c, i: (c * nb + i, 0)),
        ],
        out_shape=[
            jax.ShapeDtypeStruct((2 * _N, _D), jnp.float32),
            jax.ShapeDtypeStruct((2 * _N, _D), jnp.float32),
        ],
    )(h, _halves(Al), _halves(Vl), _halves(Bl), _halves(Bl))


def _tables2_kernel(h_ref, wa_ref, wb_ref, o_ref):
    h = h_ref[...]
    hs = jnp.dot(h, wa_ref[0], preferred_element_type=jnp.float32)
    hd = jnp.dot(h, wb_ref[0], preferred_element_type=jnp.float32)
    o_ref[...] = jnp.concatenate([hs, hd], axis=1)


def _tables2(h, Wa, Wb):
    nb = _N // _BN
    return pl.pallas_call(
        _tables2_kernel,
        grid=(2, nb),
        in_specs=[
            pl.BlockSpec((_BN, _D), lambda c, i: (i, 0)),
            pl.BlockSpec((1, _D, _DH), lambda c, i: (c, 0, 0)),
            pl.BlockSpec((1, _D, _DH), lambda c, i: (c, 0, 0)),
        ],
        out_specs=pl.BlockSpec((_BN, _D), lambda c, i: (c * nb + i, 0)),
        out_shape=jax.ShapeDtypeStruct((2 * _N, _D), jnp.float32),
    )(h, _halves(Wa), _halves(Wb))


def _node_update_kernel(h_ref, u_ref, nb_ref, acc_ref, o_ref):
    h = h_ref[...]
    a0 = acc_ref[0]
    a1 = acc_ref[1]
    agg = jnp.concatenate([a0[:, :_DH], a1[:, :_DH]], axis=1)
    den = jnp.concatenate([a0[:, _DH:], a1[:, _DH:]], axis=1) + 1e-6
    hu = jnp.dot(h, u_ref[...], preferred_element_type=jnp.float32)
    o_ref[...] = h + jnp.maximum(hu + nb_ref[...] + agg / den, 0.0)


def _node_update(h, Ul, nbl, acc3):
    return pl.pallas_call(
        _node_update_kernel,
        grid=(_N // _BN,),
        in_specs=[
            pl.BlockSpec((_BN, _D), lambda i: (i, 0)),
            pl.BlockSpec((_D, _D), lambda i: (0, 0)),
            pl.BlockSpec((1, _D), lambda i: (0, 0)),
            pl.BlockSpec((2, _BN, _D), lambda i: (0, i, 0)),
        ],
        out_specs=pl.BlockSpec((_BN, _D), lambda i: (i, 0)),
        out_shape=jax.ShapeDtypeStruct((_N, _D), jnp.float32),
    )(h, Ul, nbl.reshape(1, _D), acc3)


def _combine_kernel(p_ref, b_ref, o_ref):
    s = p_ref[0] + p_ref[1]
    o_ref[...] = jnp.sum(s, axis=1, keepdims=True) + b_ref[...]


def _combine(part3, b2sc):
    return pl.pallas_call(
        _combine_kernel,
        grid=(_E // _BE,),
        in_specs=[
            pl.BlockSpec((2, _BE, 16), lambda i: (0, i, 0)),
            pl.BlockSpec((1, 1), lambda i: (0, 0)),
        ],
        out_specs=pl.BlockSpec((_BE, 1), lambda i: (i, 0)),
        out_shape=jax.ShapeDtypeStruct((_E, 1), jnp.float32),
    )(part3, b2sc)


def _sc_edge_body(dst_h, src2_h, dst2_h, zeros_h, tsrc_h, tdst_h, efx_h,
                  efo_h, acc_h,
                  is0, is1, id0, id1, ir0, ir1, gsrc, gdst, efx, scat, efov,
                  accsh, sem0, sem1, sem_eo, sem_ix):
    c = lax.axis_index("c")
    s = lax.axis_index("s")
    cN = c * _N
    cE = c * _E
    base = s * _EPW
    sems = (sem0, sem1)
    isb = (is0, is1)
    idb = (id0, id1)
    irb = (ir0, ir1)

    pltpu.sync_copy(zeros_h.at[pl.ds(s * _ZR, _ZR)],
                    accsh.at[pl.ds(s * _ZR, _ZR)])

    @pl.when(s == 0)
    def _zero_rem():
        pltpu.sync_copy(zeros_h.at[pl.ds(_NS * _ZR, _ZREM)],
                        accsh.at[pl.ds(_NS * _ZR, _ZREM)])

    plsc.subcore_barrier()

    def _idx_issue(b, k):
        off = pl.multiple_of(base + k * _CH, 8)
        pltpu.async_copy(src2_h.at[pl.ds(cE + off, _CH)], isb[b], sem_ix)
        pltpu.async_copy(dst2_h.at[pl.ds(cE + off, _CH)], idb[b], sem_ix)

    def _issue(b, k):
        off = pl.multiple_of(base + k * _CH, 8)
        pltpu.make_async_copy(src2_h.at[pl.ds(cE + off, _CH)], isb[b],
                              sem_ix).wait()
        pltpu.make_async_copy(dst2_h.at[pl.ds(cE + off, _CH)], idb[b],
                              sem_ix).wait()
        pltpu.async_copy(tsrc_h.at[isb[b]], gsrc.at[b], sems[b])
        pltpu.async_copy(tdst_h.at[idb[b]], gdst.at[b], sems[b])
        pltpu.async_copy(efx_h.at[pl.ds(cE + off, _CH)], efx.at[b], sems[b])
        pltpu.async_copy(dst_h.at[pl.ds(off, _CH)], irb[b], sems[b])

    def _wait(b, k):
        off = pl.multiple_of(base + k * _CH, 8)
        pltpu.make_async_copy(tsrc_h.at[isb[b]], gsrc.at[b], sems[b]).wait()
        pltpu.make_async_copy(tdst_h.at[idb[b]], gdst.at[b], sems[b]).wait()
        pltpu.make_async_copy(efx_h.at[pl.ds(cE + off, _CH)], efx.at[b],
                              sems[b]).wait()
        pltpu.make_async_copy(dst_h.at[pl.ds(off, _CH)], irb[b],
                              sems[b]).wait()

    def _wait_out(kp):
        offp = pl.multiple_of(base + kp * _CH, 8)
        pltpu.make_async_copy(efov, efo_h.at[pl.ds(cE + offp, _CH)],
                              sem_eo).wait()

    def _compute(b, k):
        @pl.when(k > 0)
        def _dr():
            _wait_out(k - 1)

        def row(r, carry2):
            for jj in range(_DH // 16):
                sl = pl.ds(jj * 16, 16)
                slv = pl.ds(_DH + jj * 16, 16)
                a_ = gsrc[b, r, sl]
                v_ = gsrc[b, r, slv]
                b_ = gdst[b, r, sl]
                f_ = efx[b, r, sl]
                t_ = efx[b, r, slv]
                fn = f_ + jnp.maximum(a_ + b_ + t_, 0.0)
                efov[r, sl] = fn
                eta = 1.0 / (1.0 + jnp.exp(-fn))
                scat[r, slv] = eta
                scat[r, sl] = eta * v_
            return carry2

        lax.fori_loop(0, _CH, row, 0)
        off = pl.multiple_of(base + k * _CH, 8)
        pltpu.async_copy(efov, efo_h.at[pl.ds(cE + off, _CH)], sem_eo)
        pltpu.sync_copy(scat, accsh.at[irb[b]], add=True)

    _idx_issue(0, 0)
    _issue(0, 0)
    _idx_issue(1, 1)
    _issue(1, 1)

    def outer(t, carry):
        k = 2 * t
        _wait(0, k)

        @pl.when(k + 2 < _NCH)
        def _pf0i():
            _idx_issue(0, k + 2)

        _compute(0, k)

        @pl.when(k + 2 < _NCH)
        def _pf0():
            _issue(0, k + 2)

        _wait(1, k + 1)

        @pl.when(k + 3 < _NCH)
        def _pf1i():
            _idx_issue(1, k + 3)

        _compute(1, k + 1)

        @pl.when(k + 3 < _NCH)
        def _pf1():
            _issue(1, k + 3)

        return carry

    lax.fori_loop(0, _NCH // 2, outer, 0)
    _wait_out(_NCH - 1)
    plsc.subcore_barrier()
    pltpu.sync_copy(accsh.at[pl.ds(s * _ZR, _ZR)],
                    acc_h.at[pl.ds(cN + s * _ZR, _ZR)])

    @pl.when(s == 0)
    def _wb_rem():
        pltpu.sync_copy(accsh.at[pl.ds(_NS * _ZR, _ZREM)],
                        acc_h.at[pl.ds(cN + _NS * _ZR, _ZREM)])


@functools.cache
def _sc_edge():
    return pl.kernel(
    _sc_edge_body,
    out_type=[
        jax.ShapeDtypeStruct((2 * _E, _DH), jnp.float32),
        jax.ShapeDtypeStruct((2 * _N, _D), jnp.float32),
    ],
    mesh=plsc.VectorSubcoreMesh(core_axis_name="c", subcore_axis_name="s",
                                num_cores=2, num_subcores=_NS),
    compiler_params=pltpu.CompilerParams(needs_layout_passes=False),
    scratch_types=[
        pltpu.VMEM((_CH,), jnp.int32),
        pltpu.VMEM((_CH,), jnp.int32),
        pltpu.VMEM((_CH,), jnp.int32),
        pltpu.VMEM((_CH,), jnp.int32),
        pltpu.VMEM((_CH,), jnp.int32),
        pltpu.VMEM((_CH,), jnp.int32),
        pltpu.VMEM((2, _CH, _D), jnp.float32),
        pltpu.VMEM((2, _CH, _D), jnp.float32),
        pltpu.VMEM((2, _CH, _D), jnp.float32),
        pltpu.VMEM((_CH, _D), jnp.float32),
        pltpu.VMEM((_CH, _DH), jnp.float32),
        pltpu.VMEM_SHARED((_N, _D), jnp.float32),
        pltpu.SemaphoreType.DMA,
        pltpu.SemaphoreType.DMA,
        pltpu.SemaphoreType.DMA,
        pltpu.SemaphoreType.DMA,
    ],
    )


def _sc_score_body(src2_h, dst2_h, ts_h, efw_h, w2_h,
                   part_h,
                   is0, is1, id0, id1, g1, g2, efwv, w2v, outv,
                   sem0, sem1, sem_ix):
    c = lax.axis_index("c")
    s = lax.axis_index("s")
    cE = c * _E
    base = s * _EPW
    sems = (sem0, sem1)
    isb = (is0, is1)
    idb = (id0, id1)
    pltpu.sync_copy(w2_h.at[c], w2v)

    def _idx_issue(b, k):
        off = pl.multiple_of(base + k * _CH, 8)
        pltpu.async_copy(src2_h.at[pl.ds(cE + off, _CH)], isb[b], sem_ix)
        pltpu.async_copy(dst2_h.at[pl.ds(cE + off, _CH)], idb[b], sem_ix)

    def _issue(b, k):
        off = pl.multiple_of(base + k * _CH, 8)
        pltpu.make_async_copy(src2_h.at[pl.ds(cE + off, _CH)], isb[b],
                              sem_ix).wait()
        pltpu.make_async_copy(dst2_h.at[pl.ds(cE + off, _CH)], idb[b],
                              sem_ix).wait()
        pltpu.async_copy(ts_h.at[isb[b]], g1.at[b], sems[b])
        pltpu.async_copy(ts_h.at[idb[b]], g2.at[b], sems[b])
        pltpu.async_copy(efw_h.at[pl.ds(cE + off, _CH)], efwv.at[b], sems[b])

    def _wait(b, k):
        off = pl.multiple_of(base + k * _CH, 8)
        pltpu.make_async_copy(ts_h.at[isb[b]], g1.at[b], sems[b]).wait()
        pltpu.make_async_copy(ts_h.at[idb[b]], g2.at[b], sems[b]).wait()
        pltpu.make_async_copy(efw_h.at[pl.ds(cE + off, _CH)], efwv.at[b],
                              sems[b]).wait()

    def _compute(b, k):
        def row(r, carry2):
            acc = jnp.zeros((16,), jnp.float32)
            for jj in range(_DH // 16):
                sl = pl.ds(jj * 16, 16)
                z = (g1[b, r, sl] + g2[b, r, pl.ds(_DH + jj * 16, 16)]
                     + efwv[b, r, sl])
                acc = acc + jnp.maximum(z, 0.0) * w2v[0, sl]
            outv[r] = acc
            return carry2

        lax.fori_loop(0, _CH, row, 0)
        off = pl.multiple_of(base + k * _CH, 8)
        pltpu.sync_copy(outv, part_h.at[pl.ds(cE + off, _CH)])

    _idx_issue(0, 0)
    _issue(0, 0)
    _idx_issue(1, 1)
    _issue(1, 1)

    def outer(t, carry):
        k = 2 * t
        _wait(0, k)

        @pl.when(k + 2 < _NCH)
        def _pf0i():
            _idx_issue(0, k + 2)

        _compute(0, k)

        @pl.when(k + 2 < _NCH)
        def _pf0():
            _issue(0, k + 2)

        _wait(1, k + 1)

        @pl.when(k + 3 < _NCH)
        def _pf1i():
            _idx_issue(1, k + 3)

        _compute(1, k + 1)

        @pl.when(k + 3 < _NCH)
        def _pf1():
            _issue(1, k + 3)

        return carry

    lax.fori_loop(0, _NCH // 2, outer, 0)


@functools.cache
def _sc_score():
    return pl.kernel(
    _sc_score_body,
    out_type=jax.ShapeDtypeStruct((2 * _E, 16), jnp.float32),
    mesh=plsc.VectorSubcoreMesh(core_axis_name="c", subcore_axis_name="s",
                                num_cores=2, num_subcores=_NS),
    compiler_params=pltpu.CompilerParams(needs_layout_passes=False),
    scratch_types=[
        pltpu.VMEM((_CH,), jnp.int32),
        pltpu.VMEM((_CH,), jnp.int32),
        pltpu.VMEM((_CH,), jnp.int32),
        pltpu.VMEM((_CH,), jnp.int32),
        pltpu.VMEM((2, _CH, _D), jnp.float32),
        pltpu.VMEM((2, _CH, _D), jnp.float32),
        pltpu.VMEM((2, _CH, _DH), jnp.float32),
        pltpu.VMEM((1, _DH), jnp.float32),
        pltpu.VMEM((_CH, 16), jnp.float32),
        pltpu.SemaphoreType.DMA,
        pltpu.SemaphoreType.DMA,
        pltpu.SemaphoreType.DMA,
    ],
    )


def kernel(x, edge_index, e, Wn, bn, We, be, A, B, C, U, V, eb, nb,
           W1, b1, W2, b2):
    src = edge_index[0]
    dst = edge_index[1]
    src2 = jnp.concatenate([src, src + _N])
    dst2 = jnp.concatenate([dst, dst + _N])
    zeros = jnp.zeros((_N, _D), jnp.float32)
    h = _node_encode(x, Wn, bn)
    ef = _edge_encode(e, We, be)
    for l in range(A.shape[0]):
        tsrc, tdst = _node_mats(h, A[l], V[l], B[l])
        efx = _edge_mat_packed(ef, C[l], eb[l])
        efo, acc = _sc_edge()(dst, src2, dst2, zeros, tsrc, tdst,
                              efx.reshape(2 * _E, _D))
        ef = efo.reshape(2, _E, _DH)
        h = _node_update(h, U[l], nb[l], acc.reshape(2, _N, _D))
    ts = _tables2(h, W1[:_D], W1[_D:2 * _D])
    efw = _edge_mat(ef, W1[2 * _D:], b1)
    part = _sc_score()(src2, dst2, ts, efw.reshape(2 * _E, _DH),
                       W2[:, 0].reshape(2, 1, _DH))
    comb = _combine(part.reshape(2, _E, 16), b2.reshape(1, 1))
    return comb.reshape(_E)
